# Initial kernel scaffold; baseline (speedup 1.0000x reference)
import jax
import jax.numpy as jnp
from jax.experimental import pallas as pl


def kernel(edge_index, nfeatures, cars, free, entered, embed, W2p, b2p, W2e, b2e, W3, b3, W4, b4):
    raise NotImplementedError("write your pallas kernel here")


# speedup vs baseline: 52.4655x; 52.4655x over previous
import functools

import jax
import jax.numpy as jnp
from jax import lax
from jax.experimental import pallas as pl
from jax.experimental.pallas import tpu as pltpu
from jax.experimental.pallas import tpu_sc as plsc

N = 100000
E = 1600000
EMB = 32
VOCAB = 1000

NC = 2
NS = 16
NPT = 6272
N_PAD = NS * NPT
EPT = 50176
E_PAD = NC * NS * EPT
NCHUNKS_NODE = NPT // 128
NCHUNKS_EDGE = EPT // 128

f32 = jnp.float32
i32 = jnp.int32

_SC_PARAMS = pltpu.CompilerParams(needs_layout_passes=False,
                                  use_tc_tiling_on_sc=False)


def _fold_body(embed_ref, W2e_ref, W3_ref, W2p_ref, b2e_ref, b3_ref,
               tabP_ref, tabQ_ref, tp_ref, c0_ref):
    W3t = W3_ref[0:EMB, :]
    A = jnp.dot(W2e_ref[0:EMB, :], W3t, preferred_element_type=f32,
                precision=lax.Precision.HIGHEST)
    B = jnp.dot(W2e_ref[EMB:2 * EMB, :], W3t, preferred_element_type=f32,
                precision=lax.Precision.HIGHEST)
    emb = embed_ref[...]
    tabP_ref[...] = jnp.dot(emb, A, preferred_element_type=f32,
                            precision=lax.Precision.HIGHEST)
    tabQ_ref[...] = jnp.dot(emb, B, preferred_element_type=f32,
                            precision=lax.Precision.HIGHEST)
    tp_ref[...] = jnp.dot(emb, W2p_ref[0:EMB, :], preferred_element_type=f32,
                          precision=lax.Precision.HIGHEST)
    c0_ref[...] = jnp.dot(b2e_ref[...], W3t, preferred_element_type=f32,
                          precision=lax.Precision.HIGHEST) + b3_ref[...]


def _fold(embed, W2e, W3, W2p, b2e, b3):
    return pl.pallas_call(
        _fold_body,
        out_shape=[
            jax.ShapeDtypeStruct((VOCAB, 8), f32),
            jax.ShapeDtypeStruct((VOCAB, 8), f32),
            jax.ShapeDtypeStruct((VOCAB, 1), f32),
            jax.ShapeDtypeStruct((1, 8), f32),
        ],
    )(embed, W2e, W3, W2p, b2e.reshape(1, EMB), b3.reshape(1, 8))


def _combine_body(c_ref, e_ref, co_ref, eo_ref):
    co_ref[...] = c_ref[0, :] + c_ref[1, :]
    eo_ref[...] = e_ref[0, :] + e_ref[1, :]


def _combine(cars_part, ent_part):
    return pl.pallas_call(
        _combine_body,
        out_shape=[
            jax.ShapeDtypeStruct((N_PAD,), f32),
            jax.ShapeDtypeStruct((N_PAD,), f32),
        ],
    )(cars_part, ent_part)


_MESH = plsc.VectorSubcoreMesh(core_axis_name="c", subcore_axis_name="s",
                               num_cores=NC, num_subcores=NS)


@functools.partial(
    pl.kernel,
    out_type=[
        jax.ShapeDtypeStruct((N_PAD,), f32),
        jax.ShapeDtypeStruct((N_PAD,), f32),
        jax.ShapeDtypeStruct((N_PAD,), f32),
        jax.ShapeDtypeStruct((E_PAD,), f32),
    ],
    mesh=_MESH,
    compiler_params=_SC_PARAMS,
    scratch_types=[
        pltpu.VMEM_SHARED((N_PAD, 8), f32),
        pltpu.VMEM_SHARED((N_PAD, 8), f32),
        pltpu.VMEM_SHARED((N_PAD,), i32),
        pltpu.VMEM_SHARED((N_PAD,), f32),
        pltpu.VMEM((48,), f32),
        pltpu.VMEM((128,), i32),
        pltpu.VMEM((128,), f32),
        pltpu.VMEM((128,), f32),
        pltpu.VMEM((128,), f32),
        pltpu.VMEM((128, 8), f32),
        pltpu.VMEM((128, 8), f32),
        pltpu.VMEM((128, 8), f32),
        pltpu.VMEM((128, 8), f32),
        pltpu.VMEM((128,), f32),
        pltpu.VMEM((128,), i32),
        pltpu.VMEM((128,), i32),
        pltpu.VMEM((128,), f32),
        pltpu.VMEM((2, 128), i32),
        pltpu.SemaphoreType.DMA,
        pltpu.SemaphoreType.DMA,
        pltpu.SemaphoreType.DMA,
    ],
)
def _kern_a(nf_hbm, cars_hbm, free_hbm, ent_hbm, tabP_hbm, tabQ_hbm, tp_hbm,
            pp_hbm, src_hbm, dst_hbm,
            cars1_out, s0_out, s1_out, zs_out,
            srcP, dstQ, nfpk, Ssh, pp_v, nf_c, cars_c, free_c, ent_c,
            recS, recD, tabPg, tabQg, tpg, nfpk_c, nfd_b, cars1_c, idx2,
            sem0, sem1, sem2):
    c = lax.axis_index("c")
    s = lax.axis_index("s")
    tile = c * NS + s
    iota = lax.iota(i32, 16)

    pltpu.sync_copy(pp_hbm, pp_v)
    row0 = pp_v[pl.ds(0, 16)]
    row1 = pp_v[pl.ds(16, 16)]
    row2 = pp_v[pl.ds(32, 16)]
    wc1 = [row0[j] for j in range(8)]
    wc2 = [row0[8 + j] for j in range(8)]
    we = [row1[j] for j in range(8)]
    c0s = [row1[8 + j] for j in range(8)]
    w4s = [row2[j] for j in range(8)]
    b2p_s = row2[8]
    w2pc_s = row2[9]
    b4_s = row2[10]

    zero16 = jnp.zeros((16,), f32)

    def _zfill(i, carry):
        cars1_c[pl.ds(i * 16, 16)] = zero16
        return carry

    lax.fori_loop(0, 8, _zfill, 0)

    def _zero_loop(i, carry):
        pltpu.sync_copy(cars1_c, Ssh.at[pl.ds(s * NPT + i * 128, 128)])
        return carry

    lax.fori_loop(0, NCHUNKS_NODE, _zero_loop, 0)

    def _node_chunk(i, carry):
        nb = s * NPT + i * 128
        pltpu.sync_copy(nf_hbm.at[pl.ds(nb, 128)], nf_c)
        pltpu.sync_copy(cars_hbm.at[pl.ds(nb, 128)], cars_c)
        pltpu.sync_copy(free_hbm.at[pl.ds(nb, 128)], free_c)
        pltpu.sync_copy(ent_hbm.at[pl.ds(nb, 128)], ent_c)
        cp_p = pltpu.async_copy(tabP_hbm.at[nf_c], tabPg, sem0)
        cp_q = pltpu.async_copy(tabQ_hbm.at[nf_c], tabQg, sem1)
        cp_t = pltpu.async_copy(tp_hbm.at[nf_c], tpg, sem2)
        cp_p.wait()
        cp_q.wait()
        cp_t.wait()

        for g in range(8):
            off = g * 16
            rows = iota + off
            nfi = nf_c[pl.ds(off, 16)]
            carsv = cars_c[pl.ds(off, 16)]
            freev = free_c[pl.ds(off, 16)]
            entv = ent_c[pl.ds(off, 16)]
            tpv = tpg[pl.ds(off, 16)]
            parked = tpv + carsv * w2pc_s + b2p_s
            cars1v = jnp.maximum(jnp.maximum(parked, 0.0) + carsv, 0.0)
            for j in range(8):
                colj = jnp.full((16,), j, i32)
                pj = (plsc.load_gather(tabPg, [rows, colj])
                      + cars1v * wc1[j] + entv * we[j])
                plsc.store_scatter(recS, [rows, colj], pj)
                qj = plsc.load_gather(tabQg, [rows, colj]) + cars1v * wc2[j]
                plsc.store_scatter(recD, [rows, colj], qj)
            nfpk_c[pl.ds(off, 16)] = nfi * 2 + jnp.where(freev > 0.5, 1, 0)
            cars1_c[pl.ds(off, 16)] = cars1v

        pltpu.sync_copy(recS, srcP.at[pl.ds(nb, 128)])
        pltpu.sync_copy(recD, dstQ.at[pl.ds(nb, 128)])
        pltpu.sync_copy(nfpk_c, nfpk.at[pl.ds(nb, 128)])

        @pl.when(c == 0)
        def _():
            pltpu.sync_copy(cars1_c, cars1_out.at[pl.ds(nb, 128)])

        return carry

    lax.fori_loop(0, NCHUNKS_NODE, _node_chunk, 0)
    plsc.subcore_barrier()

    eb0 = tile * EPT

    def _edge_chunk(i, carry):
        eb = eb0 + i * 128
        pltpu.sync_copy(src_hbm.at[pl.ds(eb, 128)], idx2.at[0])
        pltpu.sync_copy(dst_hbm.at[pl.ds(eb, 128)], idx2.at[1])
        cp_s = pltpu.async_copy(srcP.at[idx2.at[0]], recS, sem0)
        cp_d = pltpu.async_copy(dstQ.at[idx2.at[1]], recD, sem1)
        cp_a = pltpu.async_copy(nfpk.at[idx2.at[0]], nfpk_c, sem2)
        cp_s.wait()
        cp_d.wait()
        cp_a.wait()
        cp_b = pltpu.async_copy(nfpk.at[idx2.at[1]], nfd_b, sem2)
        cp_b.wait()
        for g in range(8):
            off = g * 16
            rows = iota + off
            hsc = None
            for j in range(8):
                colj = jnp.full((16,), j, i32)
                ps = plsc.load_gather(recS, [rows, colj])
                qd = plsc.load_gather(recD, [rows, colj])
                rj = jnp.maximum(ps + qd + c0s[j], 0.0)
                term = rj * w4s[j]
                hsc = term if hsc is None else hsc + term
            hsc = hsc + b4_s
            a = nfpk_c[pl.ds(off, 16)]
            b = nfd_b[pl.ds(off, 16)]
            freeb = (a & 1) == 1
            selfloop = lax.shift_right_logical(a, 1) == lax.shift_right_logical(b, 1)
            enabled = selfloop != freeb
            lg = jnp.where(enabled, jnp.maximum(hsc, 0.0), 0.0)
            z = jnp.exp(lg)
            zs = jnp.where(selfloop, -z, z)
            cars_c[pl.ds(off, 16)] = z
            free_c[pl.ds(off, 16)] = zs
        pltpu.sync_copy(cars_c, Ssh.at[idx2.at[0]], add=True)
        pltpu.sync_copy(free_c, zs_out.at[pl.ds(eb, 128)])
        return carry

    lax.fori_loop(0, NCHUNKS_EDGE, _edge_chunk, 0)
    plsc.subcore_barrier()

    def _s_out(i, carry):
        nb = s * NPT + i * 128
        pltpu.sync_copy(Ssh.at[pl.ds(nb, 128)], cars1_c)

        @pl.when(c == 0)
        def _():
            pltpu.sync_copy(cars1_c, s0_out.at[pl.ds(nb, 128)])

        @pl.when(c == 1)
        def _():
            pltpu.sync_copy(cars1_c, s1_out.at[pl.ds(nb, 128)])

        return carry

    lax.fori_loop(0, NCHUNKS_NODE, _s_out, 0)


@functools.partial(
    pl.kernel,
    out_type=[
        jax.ShapeDtypeStruct((NC, N_PAD), f32),
        jax.ShapeDtypeStruct((NC, N_PAD), f32),
        jax.ShapeDtypeStruct((N_PAD, EMB), f32),
    ],
    mesh=_MESH,
    compiler_params=_SC_PARAMS,
    scratch_types=[
        pltpu.VMEM_SHARED((N_PAD,), f32),
        pltpu.VMEM_SHARED((N_PAD,), f32),
        pltpu.VMEM_SHARED((N_PAD,), f32),
        pltpu.VMEM((NPT,), f32),
        pltpu.VMEM((NPT,), f32),
        pltpu.VMEM((NPT,), f32),
        pltpu.VMEM((2, 128), i32),
        pltpu.VMEM((128,), i32),
        pltpu.VMEM((128,), f32),
        pltpu.VMEM((128,), f32),
        pltpu.VMEM((128,), f32),
        pltpu.VMEM((128,), f32),
        pltpu.VMEM((128, EMB), f32),
        pltpu.SemaphoreType.DMA,
    ],
)
def _kern_b(src_hbm, dst_hbm, zs_hbm, s0_hbm, s1_hbm, cars1_hbm, nf_hbm,
            embed_hbm,
            cars_part, ent_part, emb_out,
            T_sh, cacc, eacc, buf_a, buf_b, buf_c, idx2, nf_c, zsb, tb,
            amb, outb, embrows, sem0):
    c = lax.axis_index("c")
    s = lax.axis_index("s")
    tile = c * NS + s
    nb = s * NPT

    pltpu.sync_copy(s0_hbm.at[pl.ds(nb, NPT)], buf_a)
    pltpu.sync_copy(s1_hbm.at[pl.ds(nb, NPT)], buf_b)
    pltpu.sync_copy(cars1_hbm.at[pl.ds(nb, NPT)], buf_c)

    def _t_loop(i, carry):
        o = i * 16
        sv = buf_a[pl.ds(o, 16)] + buf_b[pl.ds(o, 16)]
        buf_a[pl.ds(o, 16)] = buf_c[pl.ds(o, 16)] / sv
        return carry

    lax.fori_loop(0, NPT // 16, _t_loop, 0)
    pltpu.sync_copy(buf_a, T_sh.at[pl.ds(nb, NPT)])

    zero16 = jnp.zeros((16,), f32)

    def _zero_loop(i, carry):
        buf_b[pl.ds(i * 16, 16)] = zero16
        return carry

    lax.fori_loop(0, NPT // 16, _zero_loop, 0)
    pltpu.sync_copy(buf_b, cacc.at[pl.ds(nb, NPT)])
    pltpu.sync_copy(buf_b, eacc.at[pl.ds(nb, NPT)])
    plsc.subcore_barrier()

    eb0 = tile * EPT

    def _edge_chunk(i, carry):
        eb = eb0 + i * 128
        pltpu.sync_copy(src_hbm.at[pl.ds(eb, 128)], idx2.at[0])
        pltpu.sync_copy(dst_hbm.at[pl.ds(eb, 128)], idx2.at[1])
        pltpu.sync_copy(zs_hbm.at[pl.ds(eb, 128)], zsb)
        pltpu.async_copy(T_sh.at[idx2.at[0]], tb, sem0).wait()
        for g in range(8):
            o = g * 16
            zsv = zsb[pl.ds(o, 16)]
            tv = tb[pl.ds(o, 16)]
            am = jnp.abs(zsv) * tv
            amb[pl.ds(o, 16)] = am
            outb[pl.ds(o, 16)] = jnp.where(zsv < 0.0, 0.0, am)
        pltpu.sync_copy(amb, cacc.at[idx2.at[1]], add=True)
        pltpu.sync_copy(outb, eacc.at[idx2.at[1]], add=True)
        return carry

    lax.fori_loop(0, NCHUNKS_EDGE, _edge_chunk, 0)
    plsc.subcore_barrier()

    pltpu.sync_copy(cacc.at[pl.ds(nb, NPT)], buf_a)
    pltpu.sync_copy(buf_a, cars_part.at[c, pl.ds(nb, NPT)])
    pltpu.sync_copy(eacc.at[pl.ds(nb, NPT)], buf_b)
    pltpu.sync_copy(buf_b, ent_part.at[c, pl.ds(nb, NPT)])

    @pl.when(c == 1)
    def _():
        def _emb_chunk(i, carry):
            nbb = s * NPT + i * 128
            pltpu.sync_copy(nf_hbm.at[pl.ds(nbb, 128)], nf_c)
            pltpu.async_copy(embed_hbm.at[nf_c], embrows, sem0).wait()
            pltpu.sync_copy(embrows, emb_out.at[pl.ds(nbb, 128)])
            return carry

        lax.fori_loop(0, NCHUNKS_NODE, _emb_chunk, 0)


def kernel(edge_index, nfeatures, cars, free, entered, embed,
           W2p, b2p, W2e, b2e, W3, b3, W4, b4):
    src = edge_index[0]
    dst = edge_index[1]
    pad_e = jnp.full((E_PAD - E,), N, i32)
    src_p = jnp.concatenate([src, pad_e])
    dst_p = jnp.concatenate([dst, pad_e])

    pad_n = N_PAD - N
    nf_p = jnp.concatenate([nfeatures, jnp.zeros((pad_n,), i32)])
    cars_p = jnp.concatenate([cars[:, 0], jnp.zeros((pad_n,), f32)])
    free_p = jnp.concatenate([free[:, 0], jnp.zeros((pad_n,), f32)])
    ent_p = jnp.concatenate([entered[:, 0], jnp.zeros((pad_n,), f32)])

    tabP, tabQ, tp, c0v = _fold(embed, W2e, W3, W2p, b2e, b3)

    pp = jnp.concatenate([
        W3[EMB], W3[EMB + 1],
        W3[EMB + 2], c0v[0],
        W4[:, 0],
        jnp.stack([b2p[0], W2p[EMB, 0], b4[0]]),
        jnp.zeros((5,), f32),
    ])

    cars1, s0, s1, zs = _kern_a(
        nf_p, cars_p, free_p, ent_p, tabP, tabQ, tp[:, 0], pp,
        src_p, dst_p)

    cars_part, ent_part, emb_out = _kern_b(
        src_p, dst_p, zs, s0, s1, cars1, nf_p, embed)

    cars_sum, ent_sum = _combine(cars_part, ent_part)

    return (cars_sum[:N, None], emb_out[:N], ent_sum[:N, None])


# speedup vs baseline: 66.5676x; 1.2688x over previous
import functools

import jax
import jax.numpy as jnp
from jax import lax
from jax.experimental import pallas as pl
from jax.experimental.pallas import tpu as pltpu
from jax.experimental.pallas import tpu_sc as plsc

N = 100000
E = 1600000
EMB = 32
VOCAB = 1000

NC = 2
NS = 16
NPT = 6272
N_PAD = NS * NPT
EPT = 50176
E_PAD = NC * NS * EPT
NCHUNKS_NODE = NPT // 128
NCHUNKS_EDGE = EPT // 128

f32 = jnp.float32
i32 = jnp.int32

_SC_PARAMS = pltpu.CompilerParams(needs_layout_passes=False,
                                  use_tc_tiling_on_sc=False)


def _fold_body(embed_ref, W2e_ref, W3_ref, W2p_ref, b2e_ref, b3_ref,
               tabP_ref, tabQ_ref, tp_ref, c0_ref):
    W3t = W3_ref[0:EMB, :]
    A = jnp.dot(W2e_ref[0:EMB, :], W3t, preferred_element_type=f32,
                precision=lax.Precision.HIGHEST)
    B = jnp.dot(W2e_ref[EMB:2 * EMB, :], W3t, preferred_element_type=f32,
                precision=lax.Precision.HIGHEST)
    emb = embed_ref[...]
    tabP_ref[...] = jnp.dot(emb, A, preferred_element_type=f32,
                            precision=lax.Precision.HIGHEST)
    tabQ_ref[...] = jnp.dot(emb, B, preferred_element_type=f32,
                            precision=lax.Precision.HIGHEST)
    tp_ref[...] = jnp.dot(emb, W2p_ref[0:EMB, :], preferred_element_type=f32,
                          precision=lax.Precision.HIGHEST)
    c0_ref[...] = jnp.dot(b2e_ref[...], W3t, preferred_element_type=f32,
                          precision=lax.Precision.HIGHEST) + b3_ref[...]


def _fold(embed, W2e, W3, W2p, b2e, b3):
    return pl.pallas_call(
        _fold_body,
        out_shape=[
            jax.ShapeDtypeStruct((VOCAB, 8), f32),
            jax.ShapeDtypeStruct((VOCAB, 8), f32),
            jax.ShapeDtypeStruct((VOCAB, 1), f32),
            jax.ShapeDtypeStruct((1, 8), f32),
        ],
    )(embed, W2e, W3, W2p, b2e.reshape(1, EMB), b3.reshape(1, 8))


def _combine_body(c_ref, e_ref, co_ref, eo_ref):
    co_ref[...] = c_ref[0, :] + c_ref[1, :]
    eo_ref[...] = e_ref[0, :] + e_ref[1, :]


def _combine(cars_part, ent_part):
    return pl.pallas_call(
        _combine_body,
        out_shape=[
            jax.ShapeDtypeStruct((N_PAD,), f32),
            jax.ShapeDtypeStruct((N_PAD,), f32),
        ],
    )(cars_part, ent_part)


_MESH = plsc.VectorSubcoreMesh(core_axis_name="c", subcore_axis_name="s",
                               num_cores=NC, num_subcores=NS)


@functools.partial(
    pl.kernel,
    out_type=[
        jax.ShapeDtypeStruct((N_PAD,), f32),
        jax.ShapeDtypeStruct((N_PAD,), f32),
        jax.ShapeDtypeStruct((N_PAD,), f32),
        jax.ShapeDtypeStruct((E_PAD,), f32),
    ],
    mesh=_MESH,
    compiler_params=_SC_PARAMS,
    scratch_types=[
        pltpu.VMEM_SHARED((N_PAD, 8), f32),
        pltpu.VMEM_SHARED((N_PAD, 8), f32),
        pltpu.VMEM_SHARED((N_PAD,), i32),
        pltpu.VMEM_SHARED((N_PAD,), f32),
        pltpu.VMEM((48,), f32),
        pltpu.VMEM((128,), i32),
        pltpu.VMEM((128,), f32),
        pltpu.VMEM((128,), f32),
        pltpu.VMEM((128,), f32),
        pltpu.VMEM((128, 8), f32),
        pltpu.VMEM((128, 8), f32),
        pltpu.VMEM((128, 8), f32),
        pltpu.VMEM((128, 8), f32),
        pltpu.VMEM((128,), f32),
        pltpu.VMEM((128,), i32),
        pltpu.VMEM((128,), i32),
        pltpu.VMEM((128,), i32),
        pltpu.VMEM((128,), i32),
        pltpu.VMEM((128,), f32),
        pltpu.VMEM((128,), i32),
        pltpu.VMEM((128,), i32),
        pltpu.VMEM((128,), i32),
        pltpu.VMEM((128,), i32),
        pltpu.VMEM((128,), i32),
        pltpu.VMEM((128,), i32),
        pltpu.VMEM((128,), f32),
        pltpu.VMEM((128,), f32),
        pltpu.VMEM((128,), f32),
        pltpu.VMEM((128,), f32),
        pltpu.SemaphoreType.DMA,
        pltpu.SemaphoreType.DMA,
        pltpu.SemaphoreType.DMA,
        pltpu.SemaphoreType.DMA,
        pltpu.SemaphoreType.DMA,
        pltpu.SemaphoreType.DMA,
    ],
)
def _kern_a(nf_hbm, cars_hbm, free_hbm, ent_hbm, tabP_hbm, tabQ_hbm, tp_hbm,
            pp_hbm, src_hbm, dst_hbm,
            cars1_out, s0_out, s1_out, zs_out,
            srcP, dstQ, nfpk, Ssh, pp_v, nf_c, cars_c, free_c, ent_c,
            recS_a, recD_a, recS_b, recD_b, tpg, nfpk_c, nfd_a, nfs_b,
            nfd_b, cars1_c, idxsrc_a, idxdst_a, idxsrc_b, idxdst_b,
            sidx_a, sidx_b, zb_a, zb_b, zsb_a, zsb_b,
            semi_a, semi_b, semg_a, semg_b, semz_a, semz_b):
    c = lax.axis_index("c")
    s = lax.axis_index("s")
    tile = c * NS + s
    iota = lax.iota(i32, 16)

    pltpu.sync_copy(pp_hbm, pp_v)
    row0 = pp_v[pl.ds(0, 16)]
    row1 = pp_v[pl.ds(16, 16)]
    row2 = pp_v[pl.ds(32, 16)]
    wc1 = [row0[j] for j in range(8)]
    wc2 = [row0[8 + j] for j in range(8)]
    we = [row1[j] for j in range(8)]
    c0s = [row1[8 + j] for j in range(8)]
    w4s = [row2[j] for j in range(8)]
    b2p_s = row2[8]
    w2pc_s = row2[9]
    b4_s = row2[10]

    zero16 = jnp.zeros((16,), f32)

    def _zfill(i, carry):
        cars1_c[pl.ds(i * 16, 16)] = zero16
        return carry

    lax.fori_loop(0, 8, _zfill, 0)

    def _zero_loop(i, carry):
        pltpu.sync_copy(cars1_c, Ssh.at[pl.ds(s * NPT + i * 128, 128)])
        return carry

    lax.fori_loop(0, NCHUNKS_NODE, _zero_loop, 0)

    def _node_chunk(i, carry):
        nb = s * NPT + i * 128
        pltpu.sync_copy(nf_hbm.at[pl.ds(nb, 128)], nf_c)
        pltpu.sync_copy(cars_hbm.at[pl.ds(nb, 128)], cars_c)
        pltpu.sync_copy(free_hbm.at[pl.ds(nb, 128)], free_c)
        pltpu.sync_copy(ent_hbm.at[pl.ds(nb, 128)], ent_c)
        cp_p = pltpu.async_copy(tabP_hbm.at[nf_c], recS_b, semi_a)
        cp_q = pltpu.async_copy(tabQ_hbm.at[nf_c], recD_b, semi_b)
        cp_t = pltpu.async_copy(tp_hbm.at[nf_c], tpg, semg_a)
        cp_p.wait()
        cp_q.wait()
        cp_t.wait()

        for g in range(8):
            off = g * 16
            rows = iota + off
            nfi = nf_c[pl.ds(off, 16)]
            carsv = cars_c[pl.ds(off, 16)]
            freev = free_c[pl.ds(off, 16)]
            entv = ent_c[pl.ds(off, 16)]
            tpv = tpg[pl.ds(off, 16)]
            parked = tpv + carsv * w2pc_s + b2p_s
            cars1v = jnp.maximum(jnp.maximum(parked, 0.0) + carsv, 0.0)
            for j in range(8):
                colj = jnp.full((16,), j, i32)
                pj = (plsc.load_gather(recS_b, [rows, colj])
                      + cars1v * wc1[j] + entv * we[j])
                plsc.store_scatter(recS_a, [rows, colj], pj)
                qj = plsc.load_gather(recD_b, [rows, colj]) + cars1v * wc2[j]
                plsc.store_scatter(recD_a, [rows, colj], qj)
            nfpk_c[pl.ds(off, 16)] = nfi * 2 + jnp.where(freev > 0.5, 1, 0)
            cars1_c[pl.ds(off, 16)] = cars1v

        pltpu.sync_copy(recS_a, srcP.at[pl.ds(nb, 128)])
        pltpu.sync_copy(recD_a, dstQ.at[pl.ds(nb, 128)])
        pltpu.sync_copy(nfpk_c, nfpk.at[pl.ds(nb, 128)])

        @pl.when(c == 0)
        def _():
            pltpu.sync_copy(cars1_c, cars1_out.at[pl.ds(nb, 128)])

        return carry

    lax.fori_loop(0, NCHUNKS_NODE, _node_chunk, 0)
    plsc.subcore_barrier()

    eb0 = tile * EPT
    KMAX = NCHUNKS_EDGE // 2

    def _load_idx(eb, isrc, idst, sem):
        pltpu.async_copy(src_hbm.at[pl.ds(eb, 128)], isrc, sem)
        pltpu.async_copy(dst_hbm.at[pl.ds(eb, 128)], idst, sem)

    def _wait_idx(eb, isrc, idst, sem):
        pltpu.make_async_copy(src_hbm.at[pl.ds(eb, 128)], isrc, sem).wait()
        pltpu.make_async_copy(dst_hbm.at[pl.ds(eb, 128)], idst, sem).wait()

    def _fire_gathers(isrc, idst, rS, rD, nS, nD, sem):
        pltpu.async_copy(srcP.at[isrc], rS, sem)
        pltpu.async_copy(dstQ.at[idst], rD, sem)
        pltpu.async_copy(nfpk.at[isrc], nS, sem)
        pltpu.async_copy(nfpk.at[idst], nD, sem)

    def _wait_gathers(isrc, idst, rS, rD, nS, nD, sem):
        pltpu.make_async_copy(srcP.at[isrc], rS, sem).wait()
        pltpu.make_async_copy(dstQ.at[idst], rD, sem).wait()
        pltpu.make_async_copy(nfpk.at[isrc], nS, sem).wait()
        pltpu.make_async_copy(nfpk.at[idst], nD, sem).wait()

    def _wait_scatter(eb, sidx, zb, zsb, sem):
        pltpu.make_async_copy(zb, Ssh.at[sidx], sem).wait()
        pltpu.make_async_copy(zsb, zs_out.at[pl.ds(eb, 128)], sem).wait()

    def _copy_sidx(isrc, sidx):
        for g in range(8):
            off = g * 16
            sidx[pl.ds(off, 16)] = isrc[pl.ds(off, 16)]

    def _compute_chunk(rS, rD, nS, nD, zb, zsb):
        for g in range(8):
            off = g * 16
            rows = iota + off
            hsc = None
            for j in range(8):
                colj = jnp.full((16,), j, i32)
                ps = plsc.load_gather(rS, [rows, colj])
                qd = plsc.load_gather(rD, [rows, colj])
                rj = jnp.maximum(ps + qd + c0s[j], 0.0)
                term = rj * w4s[j]
                hsc = term if hsc is None else hsc + term
            hsc = hsc + b4_s
            a = nS[pl.ds(off, 16)]
            b = nD[pl.ds(off, 16)]
            freeb = (a & 1) == 1
            selfloop = (lax.shift_right_logical(a, 1)
                        == lax.shift_right_logical(b, 1))
            enabled = selfloop != freeb
            lg = jnp.where(enabled, jnp.maximum(hsc, 0.0), 0.0)
            z = jnp.exp(lg)
            zs = jnp.where(selfloop, -z, z)
            zb[pl.ds(off, 16)] = z
            zsb[pl.ds(off, 16)] = zs

    _load_idx(eb0, idxsrc_a, idxdst_a, semi_a)
    _wait_idx(eb0, idxsrc_a, idxdst_a, semi_a)
    _fire_gathers(idxsrc_a, idxdst_a, recS_a, recD_a, nfpk_c, nfd_a, semg_a)
    _load_idx(eb0 + 128, idxsrc_b, idxdst_b, semi_b)

    def _edge_pair(k, carry):
        eb_a = eb0 + k * 256
        eb_b = eb_a + 128

        _wait_idx(eb_b, idxsrc_b, idxdst_b, semi_b)
        _fire_gathers(idxsrc_b, idxdst_b, recS_b, recD_b, nfs_b, nfd_b,
                      semg_b)

        _wait_gathers(idxsrc_a, idxdst_a, recS_a, recD_a, nfpk_c, nfd_a,
                      semg_a)

        _copy_sidx(idxsrc_a, sidx_a)

        @pl.when(k < KMAX - 1)
        def _():
            _load_idx(eb_a + 256, idxsrc_a, idxdst_a, semi_a)

        _compute_chunk(recS_a, recD_a, nfpk_c, nfd_a, zb_a, zsb_a)
        pltpu.sync_copy(zb_a, Ssh.at[sidx_a], add=True)
        pltpu.sync_copy(zsb_a, zs_out.at[pl.ds(eb_a, 128)])

        _wait_gathers(idxsrc_b, idxdst_b, recS_b, recD_b, nfs_b, nfd_b,
                      semg_b)

        _copy_sidx(idxsrc_b, sidx_b)

        @pl.when(k < KMAX - 1)
        def _():
            _load_idx(eb_b + 256, idxsrc_b, idxdst_b, semi_b)

        _compute_chunk(recS_b, recD_b, nfs_b, nfd_b, zb_b, zsb_b)
        pltpu.sync_copy(zb_b, Ssh.at[sidx_b], add=True)
        pltpu.sync_copy(zsb_b, zs_out.at[pl.ds(eb_b, 128)])

        @pl.when(k < KMAX - 1)
        def _():
            _wait_idx(eb_a + 256, idxsrc_a, idxdst_a, semi_a)
            _fire_gathers(idxsrc_a, idxdst_a, recS_a, recD_a, nfpk_c,
                          nfd_a, semg_a)

        return carry

    lax.fori_loop(0, KMAX, _edge_pair, 0)
    plsc.subcore_barrier()

    def _s_out(i, carry):
        nb = s * NPT + i * 128
        pltpu.sync_copy(Ssh.at[pl.ds(nb, 128)], cars1_c)

        @pl.when(c == 0)
        def _():
            pltpu.sync_copy(cars1_c, s0_out.at[pl.ds(nb, 128)])

        @pl.when(c == 1)
        def _():
            pltpu.sync_copy(cars1_c, s1_out.at[pl.ds(nb, 128)])

        return carry

    lax.fori_loop(0, NCHUNKS_NODE, _s_out, 0)


@functools.partial(
    pl.kernel,
    out_type=[
        jax.ShapeDtypeStruct((NC, N_PAD), f32),
        jax.ShapeDtypeStruct((NC, N_PAD), f32),
        jax.ShapeDtypeStruct((N_PAD, EMB), f32),
    ],
    mesh=_MESH,
    compiler_params=_SC_PARAMS,
    scratch_types=[
        pltpu.VMEM_SHARED((N_PAD,), f32),
        pltpu.VMEM_SHARED((N_PAD,), f32),
        pltpu.VMEM_SHARED((N_PAD,), f32),
        pltpu.VMEM((NPT,), f32),
        pltpu.VMEM((NPT,), f32),
        pltpu.VMEM((NPT,), f32),
        pltpu.VMEM((128,), i32),
        pltpu.VMEM((128, EMB), f32),
        pltpu.VMEM((128,), i32),
        pltpu.VMEM((128,), i32),
        pltpu.VMEM((128,), i32),
        pltpu.VMEM((128,), i32),
        pltpu.VMEM((128,), f32),
        pltpu.VMEM((128,), f32),
        pltpu.VMEM((128,), f32),
        pltpu.VMEM((128,), f32),
        pltpu.VMEM((128,), i32),
        pltpu.VMEM((128,), i32),
        pltpu.VMEM((128,), f32),
        pltpu.VMEM((128,), f32),
        pltpu.VMEM((128,), f32),
        pltpu.VMEM((128,), f32),
        pltpu.SemaphoreType.DMA,
        pltpu.SemaphoreType.DMA,
        pltpu.SemaphoreType.DMA,
        pltpu.SemaphoreType.DMA,
        pltpu.SemaphoreType.DMA,
        pltpu.SemaphoreType.DMA,
    ],
)
def _kern_b(src_hbm, dst_hbm, zs_hbm, s0_hbm, s1_hbm, cars1_hbm, nf_hbm,
            embed_hbm,
            cars_part, ent_part, emb_out,
            T_sh, cacc, eacc, buf_a, buf_b, buf_c, nf_c, embrows,
            idxsrc_a, idxdst_a, idxsrc_b, idxdst_b, zsl_a, zsl_b,
            tb_a, tb_b, sdidx_a, sdidx_b, amb_a, amb_b, outb_a, outb_b,
            semi_a, semi_b, semg_a, semg_b, semz_a, semz_b):
    c = lax.axis_index("c")
    s = lax.axis_index("s")
    tile = c * NS + s
    nb = s * NPT

    pltpu.sync_copy(s0_hbm.at[pl.ds(nb, NPT)], buf_a)
    pltpu.sync_copy(s1_hbm.at[pl.ds(nb, NPT)], buf_b)
    pltpu.sync_copy(cars1_hbm.at[pl.ds(nb, NPT)], buf_c)

    def _t_loop(i, carry):
        o = i * 16
        sv = buf_a[pl.ds(o, 16)] + buf_b[pl.ds(o, 16)]
        buf_a[pl.ds(o, 16)] = buf_c[pl.ds(o, 16)] / sv
        return carry

    lax.fori_loop(0, NPT // 16, _t_loop, 0)
    pltpu.sync_copy(buf_a, T_sh.at[pl.ds(nb, NPT)])

    zero16 = jnp.zeros((16,), f32)

    def _zero_loop(i, carry):
        buf_b[pl.ds(i * 16, 16)] = zero16
        return carry

    lax.fori_loop(0, NPT // 16, _zero_loop, 0)
    pltpu.sync_copy(buf_b, cacc.at[pl.ds(nb, NPT)])
    pltpu.sync_copy(buf_b, eacc.at[pl.ds(nb, NPT)])
    plsc.subcore_barrier()

    eb0 = tile * EPT
    KMAX = NCHUNKS_EDGE // 2

    def _load_in(eb, isrc, idst, zsl, sem):
        pltpu.async_copy(src_hbm.at[pl.ds(eb, 128)], isrc, sem)
        pltpu.async_copy(dst_hbm.at[pl.ds(eb, 128)], idst, sem)
        pltpu.async_copy(zs_hbm.at[pl.ds(eb, 128)], zsl, sem)

    def _wait_in(eb, isrc, idst, zsl, sem):
        pltpu.make_async_copy(src_hbm.at[pl.ds(eb, 128)], isrc, sem).wait()
        pltpu.make_async_copy(dst_hbm.at[pl.ds(eb, 128)], idst, sem).wait()
        pltpu.make_async_copy(zs_hbm.at[pl.ds(eb, 128)], zsl, sem).wait()

    def _wait_sc(sdidx, amb, outb, sem):
        pltpu.make_async_copy(amb, cacc.at[sdidx], sem).wait()
        pltpu.make_async_copy(outb, eacc.at[sdidx], sem).wait()

    def _compute2(zsl, tb, amb, outb):
        for g in range(8):
            o = g * 16
            zsv = zsl[pl.ds(o, 16)]
            tv = tb[pl.ds(o, 16)]
            am = jnp.abs(zsv) * tv
            amb[pl.ds(o, 16)] = am
            outb[pl.ds(o, 16)] = jnp.where(zsv < 0.0, 0.0, am)

    def _copy_sdidx(idst, sdidx):
        for g in range(8):
            o = g * 16
            sdidx[pl.ds(o, 16)] = idst[pl.ds(o, 16)]

    def _edge_chunk(i, carry):
        eb = eb0 + i * 128
        pltpu.sync_copy(src_hbm.at[pl.ds(eb, 128)], idxsrc_a)
        pltpu.sync_copy(dst_hbm.at[pl.ds(eb, 128)], idxdst_a)
        pltpu.sync_copy(zs_hbm.at[pl.ds(eb, 128)], zsl_a)
        pltpu.async_copy(T_sh.at[idxsrc_a], tb_a, semg_a).wait()
        _compute2(zsl_a, tb_a, amb_a, outb_a)
        pltpu.sync_copy(amb_a, cacc.at[idxdst_a], add=True)
        pltpu.sync_copy(outb_a, eacc.at[idxdst_a], add=True)
        return carry

    lax.fori_loop(0, NCHUNKS_EDGE, _edge_chunk, 0)
    plsc.subcore_barrier()

    pltpu.sync_copy(cacc.at[pl.ds(nb, NPT)], buf_a)
    pltpu.sync_copy(buf_a, cars_part.at[c, pl.ds(nb, NPT)])
    pltpu.sync_copy(eacc.at[pl.ds(nb, NPT)], buf_b)
    pltpu.sync_copy(buf_b, ent_part.at[c, pl.ds(nb, NPT)])

    @pl.when(c == 1)
    def _():
        def _emb_chunk(i, carry):
            nbb = s * NPT + i * 128
            pltpu.sync_copy(nf_hbm.at[pl.ds(nbb, 128)], nf_c)
            pltpu.async_copy(embed_hbm.at[nf_c], embrows, semg_a).wait()
            pltpu.sync_copy(embrows, emb_out.at[pl.ds(nbb, 128)])
            return carry

        lax.fori_loop(0, NCHUNKS_NODE, _emb_chunk, 0)


def kernel(edge_index, nfeatures, cars, free, entered, embed,
           W2p, b2p, W2e, b2e, W3, b3, W4, b4):
    src = edge_index[0]
    dst = edge_index[1]
    pad_e = jnp.full((E_PAD - E,), N, i32)
    src_p = jnp.concatenate([src, pad_e])
    dst_p = jnp.concatenate([dst, pad_e])

    pad_n = N_PAD - N
    nf_p = jnp.concatenate([nfeatures, jnp.zeros((pad_n,), i32)])
    cars_p = jnp.concatenate([cars[:, 0], jnp.zeros((pad_n,), f32)])
    free_p = jnp.concatenate([free[:, 0], jnp.zeros((pad_n,), f32)])
    ent_p = jnp.concatenate([entered[:, 0], jnp.zeros((pad_n,), f32)])

    tabP, tabQ, tp, c0v = _fold(embed, W2e, W3, W2p, b2e, b3)

    pp = jnp.concatenate([
        W3[EMB], W3[EMB + 1],
        W3[EMB + 2], c0v[0],
        W4[:, 0],
        jnp.stack([b2p[0], W2p[EMB, 0], b4[0]]),
        jnp.zeros((5,), f32),
    ])

    cars1, s0, s1, zs = _kern_a(
        nf_p, cars_p, free_p, ent_p, tabP, tabQ, tp[:, 0], pp,
        src_p, dst_p)

    cars_part, ent_part, emb_out = _kern_b(
        src_p, dst_p, zs, s0, s1, cars1, nf_p, embed)

    cars_sum, ent_sum = _combine(cars_part, ent_part)

    return (cars_sum[:N, None], emb_out[:N], ent_sum[:N, None])


# speedup vs baseline: 102.8070x; 1.5444x over previous
import functools

import jax
import jax.numpy as jnp
from jax import lax
from jax.experimental import pallas as pl
from jax.experimental.pallas import tpu as pltpu
from jax.experimental.pallas import tpu_sc as plsc

N = 100000
E = 1600000
EMB = 32
VOCAB = 1000

NC = 2
NS = 16
NPT = 6272
N_PAD = NS * NPT
EPT = 50176
E_PAD = NC * NS * EPT
NCHUNKS_NODE = NPT // 128
NCHUNKS_EDGE = EPT // 128

f32 = jnp.float32
i32 = jnp.int32

_SC_PARAMS = pltpu.CompilerParams(needs_layout_passes=False,
                                  use_tc_tiling_on_sc=False)


def _fold_body(embed_ref, W2e_ref, W3_ref, W2p_ref, b2e_ref, b3_ref,
               tabP_ref, tabQ_ref, tp_ref, c0_ref):
    W3t = W3_ref[0:EMB, :]
    A = jnp.dot(W2e_ref[0:EMB, :], W3t, preferred_element_type=f32,
                precision=lax.Precision.HIGHEST)
    B = jnp.dot(W2e_ref[EMB:2 * EMB, :], W3t, preferred_element_type=f32,
                precision=lax.Precision.HIGHEST)
    emb = embed_ref[...]
    tabP_ref[...] = jnp.dot(emb, A, preferred_element_type=f32,
                            precision=lax.Precision.HIGHEST)
    tabQ_ref[...] = jnp.dot(emb, B, preferred_element_type=f32,
                            precision=lax.Precision.HIGHEST)
    tp_ref[...] = jnp.dot(emb, W2p_ref[0:EMB, :], preferred_element_type=f32,
                          precision=lax.Precision.HIGHEST)
    c0_ref[...] = jnp.dot(b2e_ref[...], W3t, preferred_element_type=f32,
                          precision=lax.Precision.HIGHEST) + b3_ref[...]


def _fold(embed, W2e, W3, W2p, b2e, b3):
    return pl.pallas_call(
        _fold_body,
        out_shape=[
            jax.ShapeDtypeStruct((VOCAB, 8), f32),
            jax.ShapeDtypeStruct((VOCAB, 8), f32),
            jax.ShapeDtypeStruct((VOCAB, 1), f32),
            jax.ShapeDtypeStruct((1, 8), f32),
        ],
    )(embed, W2e, W3, W2p, b2e.reshape(1, EMB), b3.reshape(1, 8))


def _combine_body(c_ref, e_ref, co_ref, eo_ref):
    co_ref[...] = c_ref[0, :] + c_ref[1, :]
    eo_ref[...] = e_ref[0, :] + e_ref[1, :]


def _combine(cars_part, ent_part):
    return pl.pallas_call(
        _combine_body,
        out_shape=[
            jax.ShapeDtypeStruct((N_PAD,), f32),
            jax.ShapeDtypeStruct((N_PAD,), f32),
        ],
    )(cars_part, ent_part)


_MESH = plsc.VectorSubcoreMesh(core_axis_name="c", subcore_axis_name="s",
                               num_cores=NC, num_subcores=NS)


@functools.partial(
    pl.kernel,
    out_type=[
        jax.ShapeDtypeStruct((N_PAD,), f32),
        jax.ShapeDtypeStruct((N_PAD,), f32),
        jax.ShapeDtypeStruct((N_PAD,), f32),
        jax.ShapeDtypeStruct((E_PAD,), f32),
    ],
    mesh=_MESH,
    compiler_params=_SC_PARAMS,
    scratch_types=[
        pltpu.VMEM_SHARED((N_PAD, 8), f32),
        pltpu.VMEM_SHARED((N_PAD, 8), f32),
        pltpu.VMEM_SHARED((N_PAD,), i32),
        pltpu.VMEM_SHARED((N_PAD,), f32),
        pltpu.VMEM((48,), f32),
        pltpu.VMEM((128,), i32),
        pltpu.VMEM((128,), f32),
        pltpu.VMEM((128,), f32),
        pltpu.VMEM((128,), f32),
        pltpu.VMEM((128, 8), f32),
        pltpu.VMEM((128, 8), f32),
        pltpu.VMEM((128, 8), f32),
        pltpu.VMEM((128, 8), f32),
        pltpu.VMEM((128,), f32),
        pltpu.VMEM((128,), i32),
        pltpu.VMEM((128,), i32),
        pltpu.VMEM((128,), i32),
        pltpu.VMEM((128,), i32),
        pltpu.VMEM((128,), f32),
        pltpu.VMEM((128,), i32),
        pltpu.VMEM((128,), i32),
        pltpu.VMEM((128,), i32),
        pltpu.VMEM((128,), i32),
        pltpu.VMEM((128,), i32),
        pltpu.VMEM((128,), i32),
        pltpu.VMEM((128,), f32),
        pltpu.VMEM((128,), f32),
        pltpu.VMEM((128,), f32),
        pltpu.VMEM((128,), f32),
        pltpu.SemaphoreType.DMA,
        pltpu.SemaphoreType.DMA,
        pltpu.SemaphoreType.DMA,
        pltpu.SemaphoreType.DMA,
        pltpu.SemaphoreType.DMA,
        pltpu.SemaphoreType.DMA,
    ],
)
def _kern_a(nf_hbm, cars_hbm, free_hbm, ent_hbm, tabP_hbm, tabQ_hbm, tp_hbm,
            pp_hbm, src_hbm, dst_hbm,
            cars1_out, s0_out, s1_out, zs_out,
            srcP, dstQ, nfpk, Ssh, pp_v, nf_c, cars_c, free_c, ent_c,
            recS_a, recD_a, recS_b, recD_b, tpg, nfpk_c, nfd_a, nfs_b,
            nfd_b, cars1_c, idxsrc_a, idxdst_a, idxsrc_b, idxdst_b,
            sidx_a, sidx_b, zb_a, zb_b, zsb_a, zsb_b,
            semi_a, semi_b, semg_a, semg_b, semz_a, semz_b):
    c = lax.axis_index("c")
    s = lax.axis_index("s")
    tile = c * NS + s
    iota = lax.iota(i32, 16)

    pltpu.sync_copy(pp_hbm, pp_v)
    row0 = pp_v[pl.ds(0, 16)]
    row1 = pp_v[pl.ds(16, 16)]
    row2 = pp_v[pl.ds(32, 16)]
    wc1 = [row0[j] for j in range(8)]
    wc2 = [row0[8 + j] for j in range(8)]
    we = [row1[j] for j in range(8)]
    c0s = [row1[8 + j] for j in range(8)]
    w4s = [row2[j] for j in range(8)]
    b2p_s = row2[8]
    w2pc_s = row2[9]
    b4_s = row2[10]

    zero16 = jnp.zeros((16,), f32)

    def _zfill(i, carry):
        cars1_c[pl.ds(i * 16, 16)] = zero16
        return carry

    lax.fori_loop(0, 8, _zfill, 0)

    def _zero_loop(i, carry):
        pltpu.sync_copy(cars1_c, Ssh.at[pl.ds(s * NPT + i * 128, 128)])
        return carry

    lax.fori_loop(0, NCHUNKS_NODE, _zero_loop, 0)

    def _node_chunk(i, carry):
        nb = s * NPT + i * 128
        pltpu.sync_copy(nf_hbm.at[pl.ds(nb, 128)], nf_c)
        pltpu.sync_copy(cars_hbm.at[pl.ds(nb, 128)], cars_c)
        pltpu.sync_copy(free_hbm.at[pl.ds(nb, 128)], free_c)
        pltpu.sync_copy(ent_hbm.at[pl.ds(nb, 128)], ent_c)
        cp_p = pltpu.async_copy(tabP_hbm.at[nf_c], recS_b, semi_a)
        cp_q = pltpu.async_copy(tabQ_hbm.at[nf_c], recD_b, semi_b)
        cp_t = pltpu.async_copy(tp_hbm.at[nf_c], tpg, semg_a)
        cp_p.wait()
        cp_q.wait()
        cp_t.wait()

        for g in range(8):
            off = g * 16
            rows = iota + off
            nfi = nf_c[pl.ds(off, 16)]
            carsv = cars_c[pl.ds(off, 16)]
            freev = free_c[pl.ds(off, 16)]
            entv = ent_c[pl.ds(off, 16)]
            tpv = tpg[pl.ds(off, 16)]
            parked = tpv + carsv * w2pc_s + b2p_s
            cars1v = jnp.maximum(jnp.maximum(parked, 0.0) + carsv, 0.0)
            for j in range(8):
                colj = jnp.full((16,), j, i32)
                pj = (plsc.load_gather(recS_b, [rows, colj])
                      + cars1v * wc1[j] + entv * we[j])
                plsc.store_scatter(recS_a, [rows, colj], pj)
                qj = plsc.load_gather(recD_b, [rows, colj]) + cars1v * wc2[j]
                plsc.store_scatter(recD_a, [rows, colj], qj)
            nfpk_c[pl.ds(off, 16)] = nfi * 2 + jnp.where(freev > 0.5, 1, 0)
            cars1_c[pl.ds(off, 16)] = cars1v

        pltpu.sync_copy(recS_a, srcP.at[pl.ds(nb, 128)])
        pltpu.sync_copy(recD_a, dstQ.at[pl.ds(nb, 128)])
        pltpu.sync_copy(nfpk_c, nfpk.at[pl.ds(nb, 128)])

        @pl.when(c == 0)
        def _():
            pltpu.sync_copy(cars1_c, cars1_out.at[pl.ds(nb, 128)])

        return carry

    lax.fori_loop(0, NCHUNKS_NODE, _node_chunk, 0)
    plsc.subcore_barrier()

    eb0 = tile * EPT
    KMAX = NCHUNKS_EDGE // 2

    def _load_idx(eb, isrc, idst, sem):
        pltpu.async_copy(src_hbm.at[pl.ds(eb, 128)], isrc, sem)
        pltpu.async_copy(dst_hbm.at[pl.ds(eb, 128)], idst, sem)

    def _wait_idx(eb, isrc, idst, sem):
        pltpu.make_async_copy(src_hbm.at[pl.ds(eb, 128)], isrc, sem).wait()
        pltpu.make_async_copy(dst_hbm.at[pl.ds(eb, 128)], idst, sem).wait()

    def _fire_gathers(isrc, idst, rS, rD, nS, nD, sem):
        pltpu.async_copy(srcP.at[isrc], rS, sem)
        pltpu.async_copy(dstQ.at[idst], rD, sem)
        pltpu.async_copy(nfpk.at[isrc], nS, sem)
        pltpu.async_copy(nfpk.at[idst], nD, sem)

    def _wait_gathers(isrc, idst, rS, rD, nS, nD, sem):
        pltpu.make_async_copy(srcP.at[isrc], rS, sem).wait()
        pltpu.make_async_copy(dstQ.at[idst], rD, sem).wait()
        pltpu.make_async_copy(nfpk.at[isrc], nS, sem).wait()
        pltpu.make_async_copy(nfpk.at[idst], nD, sem).wait()

    def _wait_scatter(eb, sidx, zb, zsb, sem):
        pltpu.make_async_copy(zb, Ssh.at[sidx], sem).wait()
        pltpu.make_async_copy(zsb, zs_out.at[pl.ds(eb, 128)], sem).wait()

    def _copy_sidx(isrc, sidx):
        for g in range(8):
            off = g * 16
            sidx[pl.ds(off, 16)] = isrc[pl.ds(off, 16)]

    def _compute_chunk(rS, rD, nS, nD, zb, zsb):
        for g in range(8):
            off = g * 16
            rows = iota + off
            hsc = None
            for j in range(8):
                colj = jnp.full((16,), j, i32)
                ps = plsc.load_gather(rS, [rows, colj])
                qd = plsc.load_gather(rD, [rows, colj])
                rj = jnp.maximum(ps + qd + c0s[j], 0.0)
                term = rj * w4s[j]
                hsc = term if hsc is None else hsc + term
            hsc = hsc + b4_s
            a = nS[pl.ds(off, 16)]
            b = nD[pl.ds(off, 16)]
            freeb = (a & 1) == 1
            selfloop = (lax.shift_right_logical(a, 1)
                        == lax.shift_right_logical(b, 1))
            enabled = selfloop != freeb
            lg = jnp.where(enabled, jnp.maximum(hsc, 0.0), 0.0)
            z = jnp.exp(lg)
            zs = jnp.where(selfloop, -z, z)
            zb[pl.ds(off, 16)] = z
            zsb[pl.ds(off, 16)] = zs

    _load_idx(eb0, idxsrc_a, idxdst_a, semi_a)
    _wait_idx(eb0, idxsrc_a, idxdst_a, semi_a)
    _fire_gathers(idxsrc_a, idxdst_a, recS_a, recD_a, nfpk_c, nfd_a, semg_a)
    _load_idx(eb0 + 128, idxsrc_b, idxdst_b, semi_b)

    def _edge_pair(k, carry):
        eb_a = eb0 + k * 256
        eb_b = eb_a + 128

        _wait_idx(eb_b, idxsrc_b, idxdst_b, semi_b)
        _fire_gathers(idxsrc_b, idxdst_b, recS_b, recD_b, nfs_b, nfd_b,
                      semg_b)

        _wait_gathers(idxsrc_a, idxdst_a, recS_a, recD_a, nfpk_c, nfd_a,
                      semg_a)

        _copy_sidx(idxsrc_a, sidx_a)

        @pl.when(k < KMAX - 1)
        def _():
            _load_idx(eb_a + 256, idxsrc_a, idxdst_a, semi_a)

        _compute_chunk(recS_a, recD_a, nfpk_c, nfd_a, zb_a, zsb_a)
        pltpu.sync_copy(zb_a, Ssh.at[sidx_a], add=True)
        pltpu.sync_copy(zsb_a, zs_out.at[pl.ds(eb_a, 128)])

        _wait_gathers(idxsrc_b, idxdst_b, recS_b, recD_b, nfs_b, nfd_b,
                      semg_b)

        _copy_sidx(idxsrc_b, sidx_b)

        @pl.when(k < KMAX - 1)
        def _():
            _load_idx(eb_b + 256, idxsrc_b, idxdst_b, semi_b)

        _compute_chunk(recS_b, recD_b, nfs_b, nfd_b, zb_b, zsb_b)
        pltpu.sync_copy(zb_b, Ssh.at[sidx_b], add=True)
        pltpu.sync_copy(zsb_b, zs_out.at[pl.ds(eb_b, 128)])

        @pl.when(k < KMAX - 1)
        def _():
            _wait_idx(eb_a + 256, idxsrc_a, idxdst_a, semi_a)
            _fire_gathers(idxsrc_a, idxdst_a, recS_a, recD_a, nfpk_c,
                          nfd_a, semg_a)

        return carry

    lax.fori_loop(0, KMAX, _edge_pair, 0)
    plsc.subcore_barrier()

    def _s_out(i, carry):
        nb = s * NPT + i * 128
        pltpu.sync_copy(Ssh.at[pl.ds(nb, 128)], cars1_c)

        @pl.when(c == 0)
        def _():
            pltpu.sync_copy(cars1_c, s0_out.at[pl.ds(nb, 128)])

        @pl.when(c == 1)
        def _():
            pltpu.sync_copy(cars1_c, s1_out.at[pl.ds(nb, 128)])

        return carry

    lax.fori_loop(0, NCHUNKS_NODE, _s_out, 0)


@functools.partial(
    pl.kernel,
    out_type=[
        jax.ShapeDtypeStruct((NC, N_PAD), f32),
        jax.ShapeDtypeStruct((NC, N_PAD), f32),
        jax.ShapeDtypeStruct((N_PAD, EMB), f32),
    ],
    mesh=_MESH,
    compiler_params=_SC_PARAMS,
    scratch_types=[
        pltpu.VMEM_SHARED((N_PAD,), f32),
        pltpu.VMEM_SHARED((N_PAD,), f32),
        pltpu.VMEM_SHARED((N_PAD,), f32),
        pltpu.VMEM((NPT,), f32),
        pltpu.VMEM((NPT,), f32),
        pltpu.VMEM((NPT,), f32),
        pltpu.VMEM((128,), i32),
        pltpu.VMEM((128, EMB), f32),
        pltpu.VMEM((128,), i32),
        pltpu.VMEM((128,), i32),
        pltpu.VMEM((128,), i32),
        pltpu.VMEM((128,), i32),
        pltpu.VMEM((128,), f32),
        pltpu.VMEM((128,), f32),
        pltpu.VMEM((128,), f32),
        pltpu.VMEM((128,), f32),
        pltpu.VMEM((128,), i32),
        pltpu.VMEM((128,), i32),
        pltpu.VMEM((128,), f32),
        pltpu.VMEM((128,), f32),
        pltpu.VMEM((128,), f32),
        pltpu.VMEM((128,), f32),
        pltpu.SemaphoreType.DMA,
        pltpu.SemaphoreType.DMA,
        pltpu.SemaphoreType.DMA,
        pltpu.SemaphoreType.DMA,
        pltpu.SemaphoreType.DMA,
        pltpu.SemaphoreType.DMA,
    ],
)
def _kern_b(src_hbm, dst_hbm, zs_hbm, s0_hbm, s1_hbm, cars1_hbm, nf_hbm,
            embed_hbm,
            cars_part, ent_part, emb_out,
            T_sh, cacc, eacc, buf_a, buf_b, buf_c, nf_c, embrows,
            idxsrc_a, idxdst_a, idxsrc_b, idxdst_b, zsl_a, zsl_b,
            tb_a, tb_b, sdidx_a, sdidx_b, amb_a, amb_b, outb_a, outb_b,
            semi_a, semi_b, semg_a, semg_b, semz_a, semz_b):
    c = lax.axis_index("c")
    s = lax.axis_index("s")
    tile = c * NS + s
    nb = s * NPT

    pltpu.sync_copy(s0_hbm.at[pl.ds(nb, NPT)], buf_a)
    pltpu.sync_copy(s1_hbm.at[pl.ds(nb, NPT)], buf_b)
    pltpu.sync_copy(cars1_hbm.at[pl.ds(nb, NPT)], buf_c)

    def _t_loop(i, carry):
        o = i * 16
        sv = buf_a[pl.ds(o, 16)] + buf_b[pl.ds(o, 16)]
        buf_a[pl.ds(o, 16)] = buf_c[pl.ds(o, 16)] / sv
        return carry

    lax.fori_loop(0, NPT // 16, _t_loop, 0)
    pltpu.sync_copy(buf_a, T_sh.at[pl.ds(nb, NPT)])

    zero16 = jnp.zeros((16,), f32)

    def _zero_loop(i, carry):
        buf_b[pl.ds(i * 16, 16)] = zero16
        return carry

    lax.fori_loop(0, NPT // 16, _zero_loop, 0)
    pltpu.sync_copy(buf_b, cacc.at[pl.ds(nb, NPT)])
    pltpu.sync_copy(buf_b, eacc.at[pl.ds(nb, NPT)])
    plsc.subcore_barrier()

    eb0 = tile * EPT
    KMAX = NCHUNKS_EDGE // 2

    def _load_in(eb, isrc, idst, zsl, sem):
        pltpu.async_copy(src_hbm.at[pl.ds(eb, 128)], isrc, sem)
        pltpu.async_copy(dst_hbm.at[pl.ds(eb, 128)], idst, sem)
        pltpu.async_copy(zs_hbm.at[pl.ds(eb, 128)], zsl, sem)

    def _wait_in(eb, isrc, idst, zsl, sem):
        pltpu.make_async_copy(src_hbm.at[pl.ds(eb, 128)], isrc, sem).wait()
        pltpu.make_async_copy(dst_hbm.at[pl.ds(eb, 128)], idst, sem).wait()
        pltpu.make_async_copy(zs_hbm.at[pl.ds(eb, 128)], zsl, sem).wait()

    def _wait_sc(sdidx, amb, outb, sem):
        pltpu.make_async_copy(amb, cacc.at[sdidx], sem).wait()
        pltpu.make_async_copy(outb, eacc.at[sdidx], sem).wait()

    def _compute2(zsl, tb, amb, outb):
        for g in range(8):
            o = g * 16
            zsv = zsl[pl.ds(o, 16)]
            tv = tb[pl.ds(o, 16)]
            am = jnp.abs(zsv) * tv
            amb[pl.ds(o, 16)] = am
            outb[pl.ds(o, 16)] = jnp.where(zsv < 0.0, 0.0, am)

    def _copy_sdidx(idst, sdidx):
        for g in range(8):
            o = g * 16
            sdidx[pl.ds(o, 16)] = idst[pl.ds(o, 16)]

    _load_in(eb0, idxsrc_a, idxdst_a, zsl_a, semi_a)
    _wait_in(eb0, idxsrc_a, idxdst_a, zsl_a, semi_a)
    pltpu.async_copy(T_sh.at[idxsrc_a], tb_a, semg_a)
    _load_in(eb0 + 128, idxsrc_b, idxdst_b, zsl_b, semi_b)

    def _edge_pair(k, carry):
        eb_a = eb0 + k * 256
        eb_b = eb_a + 128

        _wait_in(eb_b, idxsrc_b, idxdst_b, zsl_b, semi_b)
        pltpu.async_copy(T_sh.at[idxsrc_b], tb_b, semg_b)

        pltpu.make_async_copy(T_sh.at[idxsrc_a], tb_a, semg_a).wait()
        _compute2(zsl_a, tb_a, amb_a, outb_a)
        _copy_sdidx(idxdst_a, sdidx_a)

        @pl.when(k < KMAX - 1)
        def _():
            _load_in(eb_a + 256, idxsrc_a, idxdst_a, zsl_a, semi_a)

        pltpu.sync_copy(amb_a, cacc.at[sdidx_a], add=True)
        pltpu.sync_copy(outb_a, eacc.at[sdidx_a], add=True)

        pltpu.make_async_copy(T_sh.at[idxsrc_b], tb_b, semg_b).wait()
        _compute2(zsl_b, tb_b, amb_b, outb_b)
        _copy_sdidx(idxdst_b, sdidx_b)

        @pl.when(k < KMAX - 1)
        def _():
            _load_in(eb_b + 256, idxsrc_b, idxdst_b, zsl_b, semi_b)

        pltpu.sync_copy(amb_b, cacc.at[sdidx_b], add=True)
        pltpu.sync_copy(outb_b, eacc.at[sdidx_b], add=True)

        @pl.when(k < KMAX - 1)
        def _():
            _wait_in(eb_a + 256, idxsrc_a, idxdst_a, zsl_a, semi_a)
            pltpu.async_copy(T_sh.at[idxsrc_a], tb_a, semg_a)

        return carry

    lax.fori_loop(0, KMAX, _edge_pair, 0)
    plsc.subcore_barrier()

    pltpu.sync_copy(cacc.at[pl.ds(nb, NPT)], buf_a)
    pltpu.sync_copy(buf_a, cars_part.at[c, pl.ds(nb, NPT)])
    pltpu.sync_copy(eacc.at[pl.ds(nb, NPT)], buf_b)
    pltpu.sync_copy(buf_b, ent_part.at[c, pl.ds(nb, NPT)])

    @pl.when(c == 1)
    def _():
        def _emb_chunk(i, carry):
            nbb = s * NPT + i * 128
            pltpu.sync_copy(nf_hbm.at[pl.ds(nbb, 128)], nf_c)
            pltpu.async_copy(embed_hbm.at[nf_c], embrows, semg_a).wait()
            pltpu.sync_copy(embrows, emb_out.at[pl.ds(nbb, 128)])
            return carry

        lax.fori_loop(0, NCHUNKS_NODE, _emb_chunk, 0)


def kernel(edge_index, nfeatures, cars, free, entered, embed,
           W2p, b2p, W2e, b2e, W3, b3, W4, b4):
    src = edge_index[0]
    dst = edge_index[1]
    pad_e = jnp.full((E_PAD - E,), N, i32)
    src_p = jnp.concatenate([src, pad_e])
    dst_p = jnp.concatenate([dst, pad_e])

    pad_n = N_PAD - N
    nf_p = jnp.concatenate([nfeatures, jnp.zeros((pad_n,), i32)])
    cars_p = jnp.concatenate([cars[:, 0], jnp.zeros((pad_n,), f32)])
    free_p = jnp.concatenate([free[:, 0], jnp.zeros((pad_n,), f32)])
    ent_p = jnp.concatenate([entered[:, 0], jnp.zeros((pad_n,), f32)])

    tabP, tabQ, tp, c0v = _fold(embed, W2e, W3, W2p, b2e, b3)

    pp = jnp.concatenate([
        W3[EMB], W3[EMB + 1],
        W3[EMB + 2], c0v[0],
        W4[:, 0],
        jnp.stack([b2p[0], W2p[EMB, 0], b4[0]]),
        jnp.zeros((5,), f32),
    ])

    cars1, s0, s1, zs = _kern_a(
        nf_p, cars_p, free_p, ent_p, tabP, tabQ, tp[:, 0], pp,
        src_p, dst_p)

    cars_part, ent_part, emb_out = _kern_b(
        src_p, dst_p, zs, s0, s1, cars1, nf_p, embed)

    cars_sum, ent_sum = _combine(cars_part, ent_part)

    return (cars_sum[:N, None], emb_out[:N], ent_sum[:N, None])


# speedup vs baseline: 106.1718x; 1.0327x over previous
import functools

import jax
import jax.numpy as jnp
from jax import lax
from jax.experimental import pallas as pl
from jax.experimental.pallas import tpu as pltpu
from jax.experimental.pallas import tpu_sc as plsc

N = 100000
E = 1600000
EMB = 32
VOCAB = 1000

NC = 2
NS = 16
NPT = 6272
N_PAD = NS * NPT
EPT = 50176
E_PAD = NC * NS * EPT
NCHUNKS_NODE = NPT // 128
NCHUNKS_EDGE = EPT // 128

f32 = jnp.float32
i32 = jnp.int32

_SC_PARAMS = pltpu.CompilerParams(needs_layout_passes=False,
                                  use_tc_tiling_on_sc=False)


def _fold_body(embed_ref, W2e_ref, W3_ref, W2p_ref, b2e_ref, b3_ref,
               tabP_ref, tabQ_ref, tp_ref, c0_ref):
    W3t = W3_ref[0:EMB, :]
    A = jnp.dot(W2e_ref[0:EMB, :], W3t, preferred_element_type=f32,
                precision=lax.Precision.HIGHEST)
    B = jnp.dot(W2e_ref[EMB:2 * EMB, :], W3t, preferred_element_type=f32,
                precision=lax.Precision.HIGHEST)
    emb = embed_ref[...]
    tabP_ref[...] = jnp.dot(emb, A, preferred_element_type=f32,
                            precision=lax.Precision.HIGHEST)
    tabQ_ref[...] = jnp.dot(emb, B, preferred_element_type=f32,
                            precision=lax.Precision.HIGHEST)
    tp_ref[...] = jnp.dot(emb, W2p_ref[0:EMB, :], preferred_element_type=f32,
                          precision=lax.Precision.HIGHEST)
    c0_ref[...] = jnp.dot(b2e_ref[...], W3t, preferred_element_type=f32,
                          precision=lax.Precision.HIGHEST) + b3_ref[...]


def _fold(embed, W2e, W3, W2p, b2e, b3):
    return pl.pallas_call(
        _fold_body,
        out_shape=[
            jax.ShapeDtypeStruct((VOCAB, 8), f32),
            jax.ShapeDtypeStruct((VOCAB, 8), f32),
            jax.ShapeDtypeStruct((VOCAB, 1), f32),
            jax.ShapeDtypeStruct((1, 8), f32),
        ],
    )(embed, W2e, W3, W2p, b2e.reshape(1, EMB), b3.reshape(1, 8))


def _combine_body(c_ref, e_ref, co_ref, eo_ref):
    co_ref[...] = c_ref[0, :] + c_ref[1, :]
    eo_ref[...] = e_ref[0, :] + e_ref[1, :]


def _combine(cars_part, ent_part):
    return pl.pallas_call(
        _combine_body,
        out_shape=[
            jax.ShapeDtypeStruct((N_PAD,), f32),
            jax.ShapeDtypeStruct((N_PAD,), f32),
        ],
    )(cars_part, ent_part)


_MESH = plsc.VectorSubcoreMesh(core_axis_name="c", subcore_axis_name="s",
                               num_cores=NC, num_subcores=NS)


@functools.partial(
    pl.kernel,
    out_type=[
        jax.ShapeDtypeStruct((N_PAD,), f32),
        jax.ShapeDtypeStruct((N_PAD,), f32),
        jax.ShapeDtypeStruct((N_PAD,), f32),
        jax.ShapeDtypeStruct((E_PAD,), f32),
    ],
    mesh=_MESH,
    compiler_params=_SC_PARAMS,
    scratch_types=[
        pltpu.VMEM_SHARED((N_PAD, 8), f32),
        pltpu.VMEM_SHARED((N_PAD, 8), f32),
        pltpu.VMEM_SHARED((N_PAD,), i32),
        pltpu.VMEM_SHARED((N_PAD,), f32),
        pltpu.VMEM((48,), f32),
        pltpu.VMEM((128,), i32),
        pltpu.VMEM((128,), f32),
        pltpu.VMEM((128,), f32),
        pltpu.VMEM((128,), f32),
        pltpu.VMEM((128, 8), f32),
        pltpu.VMEM((128, 8), f32),
        pltpu.VMEM((128, 8), f32),
        pltpu.VMEM((128, 8), f32),
        pltpu.VMEM((128,), f32),
        pltpu.VMEM((128,), i32),
        pltpu.VMEM((128,), i32),
        pltpu.VMEM((128,), i32),
        pltpu.VMEM((128,), i32),
        pltpu.VMEM((128,), f32),
        pltpu.VMEM((128,), i32),
        pltpu.VMEM((128,), i32),
        pltpu.VMEM((128,), i32),
        pltpu.VMEM((128,), i32),
        pltpu.VMEM((128,), i32),
        pltpu.VMEM((128,), i32),
        pltpu.VMEM((128,), f32),
        pltpu.VMEM((128,), f32),
        pltpu.VMEM((128,), f32),
        pltpu.VMEM((128,), f32),
        pltpu.VMEM((128,), i32),
        pltpu.VMEM((128,), f32),
        pltpu.VMEM((128,), f32),
        pltpu.VMEM((128,), f32),
        pltpu.VMEM((128, 8), f32),
        pltpu.VMEM((128, 8), f32),
        pltpu.SemaphoreType.DMA,
        pltpu.SemaphoreType.DMA,
        pltpu.SemaphoreType.DMA,
        pltpu.SemaphoreType.DMA,
        pltpu.SemaphoreType.DMA,
        pltpu.SemaphoreType.DMA,
    ],
)
def _kern_a(nf_hbm, cars_hbm, free_hbm, ent_hbm, tabP_hbm, tabQ_hbm, tp_hbm,
            pp_hbm, src_hbm, dst_hbm,
            cars1_out, s0_out, s1_out, zs_out,
            srcP, dstQ, nfpk, Ssh, pp_v, nf_c, cars_c, free_c, ent_c,
            recS_a, recD_a, recS_b, recD_b, tpg, nfpk_c, nfd_a, nfs_b,
            nfd_b, cars1_c, idxsrc_a, idxdst_a, idxsrc_b, idxdst_b,
            sidx_a, sidx_b, zb_a, zb_b, zsb_a, zsb_b,
            nfl_b, carl_b, frel_b, entl_b, recW_s, recW_d,
            semi_a, semi_b, semg_a, semg_b, semz_a, semz_b):
    c = lax.axis_index("c")
    s = lax.axis_index("s")
    tile = c * NS + s
    iota = lax.iota(i32, 16)

    pltpu.sync_copy(pp_hbm, pp_v)
    row0 = pp_v[pl.ds(0, 16)]
    row1 = pp_v[pl.ds(16, 16)]
    row2 = pp_v[pl.ds(32, 16)]
    wc1 = [row0[j] for j in range(8)]
    wc2 = [row0[8 + j] for j in range(8)]
    we = [row1[j] for j in range(8)]
    c0s = [row1[8 + j] for j in range(8)]
    w4s = [row2[j] for j in range(8)]
    b2p_s = row2[8]
    w2pc_s = row2[9]
    b4_s = row2[10]

    zero16 = jnp.zeros((16,), f32)

    def _zfill(i, carry):
        cars1_c[pl.ds(i * 16, 16)] = zero16
        return carry

    lax.fori_loop(0, 8, _zfill, 0)

    def _zero_loop(i, carry):
        pltpu.sync_copy(cars1_c, Ssh.at[pl.ds(s * NPT + i * 128, 128)])
        return carry

    lax.fori_loop(0, NCHUNKS_NODE, _zero_loop, 0)

    nb0 = s * NPT

    def _n_load(nb, nfb, carb, freb, entb, sem):
        pltpu.async_copy(nf_hbm.at[pl.ds(nb, 128)], nfb, sem)
        pltpu.async_copy(cars_hbm.at[pl.ds(nb, 128)], carb, sem)
        pltpu.async_copy(free_hbm.at[pl.ds(nb, 128)], freb, sem)
        pltpu.async_copy(ent_hbm.at[pl.ds(nb, 128)], entb, sem)

    def _n_wait_load(nb, nfb, carb, freb, entb, sem):
        pltpu.make_async_copy(nf_hbm.at[pl.ds(nb, 128)], nfb, sem).wait()
        pltpu.make_async_copy(cars_hbm.at[pl.ds(nb, 128)], carb, sem).wait()
        pltpu.make_async_copy(free_hbm.at[pl.ds(nb, 128)], freb, sem).wait()
        pltpu.make_async_copy(ent_hbm.at[pl.ds(nb, 128)], entb, sem).wait()

    def _n_gather(nfb, tPg, tQg, tpb, sem):
        pltpu.async_copy(tabP_hbm.at[nfb], tPg, sem)
        pltpu.async_copy(tabQ_hbm.at[nfb], tQg, sem)
        pltpu.async_copy(tp_hbm.at[nfb], tpb, sem)

    def _n_wait_gather(nfb, tPg, tQg, tpb, sem):
        pltpu.make_async_copy(tabP_hbm.at[nfb], tPg, sem).wait()
        pltpu.make_async_copy(tabQ_hbm.at[nfb], tQg, sem).wait()
        pltpu.make_async_copy(tp_hbm.at[nfb], tpb, sem).wait()

    def _n_compute(nb, nfb, carb, freb, entb, tPg, tQg, tpb):
        for g in range(8):
            off = g * 16
            rows = iota + off
            nfi = nfb[pl.ds(off, 16)]
            carsv = carb[pl.ds(off, 16)]
            freev = freb[pl.ds(off, 16)]
            entv = entb[pl.ds(off, 16)]
            tpv = tpb[pl.ds(off, 16)]
            parked = tpv + carsv * w2pc_s + b2p_s
            cars1v = jnp.maximum(jnp.maximum(parked, 0.0) + carsv, 0.0)
            for j in range(8):
                colj = jnp.full((16,), j, i32)
                pj = (plsc.load_gather(tPg, [rows, colj])
                      + cars1v * wc1[j] + entv * we[j])
                plsc.store_scatter(recW_s, [rows, colj], pj)
                qj = plsc.load_gather(tQg, [rows, colj]) + cars1v * wc2[j]
                plsc.store_scatter(recW_d, [rows, colj], qj)
            nfpk_c[pl.ds(off, 16)] = nfi * 2 + jnp.where(freev > 0.5, 1, 0)
            cars1_c[pl.ds(off, 16)] = cars1v
        pltpu.sync_copy(recW_s, srcP.at[pl.ds(nb, 128)])
        pltpu.sync_copy(recW_d, dstQ.at[pl.ds(nb, 128)])
        pltpu.sync_copy(nfpk_c, nfpk.at[pl.ds(nb, 128)])

        @pl.when(c == 0)
        def _():
            pltpu.sync_copy(cars1_c, cars1_out.at[pl.ds(nb, 128)])

    _n_load(nb0, nf_c, cars_c, free_c, ent_c, semi_a)
    _n_wait_load(nb0, nf_c, cars_c, free_c, ent_c, semi_a)
    _n_gather(nf_c, recS_a, recD_a, zb_a, semg_a)
    _n_load(nb0 + 128, nfl_b, carl_b, frel_b, entl_b, semi_b)

    def _node_pair(k, carry):
        nb_a = nb0 + k * 256
        nb_b = nb_a + 128

        _n_wait_load(nb_b, nfl_b, carl_b, frel_b, entl_b, semi_b)
        _n_gather(nfl_b, recS_b, recD_b, zb_b, semg_b)

        _n_wait_gather(nf_c, recS_a, recD_a, zb_a, semg_a)
        _n_compute(nb_a, nf_c, cars_c, free_c, ent_c, recS_a, recD_a, zb_a)

        _n_load(nb_a + 256, nf_c, cars_c, free_c, ent_c, semi_a)

        _n_wait_gather(nfl_b, recS_b, recD_b, zb_b, semg_b)
        _n_compute(nb_b, nfl_b, carl_b, frel_b, entl_b, recS_b, recD_b,
                   zb_b)

        @pl.when(k < 23)
        def _():
            _n_load(nb_b + 256, nfl_b, carl_b, frel_b, entl_b, semi_b)

        _n_wait_load(nb_a + 256, nf_c, cars_c, free_c, ent_c, semi_a)
        _n_gather(nf_c, recS_a, recD_a, zb_a, semg_a)
        return carry

    lax.fori_loop(0, (NCHUNKS_NODE - 1) // 2, _node_pair, 0)
    _n_wait_gather(nf_c, recS_a, recD_a, zb_a, semg_a)
    _n_compute(nb0 + (NCHUNKS_NODE - 1) * 128, nf_c, cars_c, free_c,
               ent_c, recS_a, recD_a, zb_a)
    plsc.subcore_barrier()

    eb0 = tile * EPT
    KMAX = NCHUNKS_EDGE // 2

    def _load_idx(eb, isrc, idst, sem):
        pltpu.async_copy(src_hbm.at[pl.ds(eb, 128)], isrc, sem)
        pltpu.async_copy(dst_hbm.at[pl.ds(eb, 128)], idst, sem)

    def _wait_idx(eb, isrc, idst, sem):
        pltpu.make_async_copy(src_hbm.at[pl.ds(eb, 128)], isrc, sem).wait()
        pltpu.make_async_copy(dst_hbm.at[pl.ds(eb, 128)], idst, sem).wait()

    def _fire_gathers(isrc, idst, rS, rD, nS, nD, sem):
        pltpu.async_copy(srcP.at[isrc], rS, sem)
        pltpu.async_copy(dstQ.at[idst], rD, sem)
        pltpu.async_copy(nfpk.at[isrc], nS, sem)
        pltpu.async_copy(nfpk.at[idst], nD, sem)

    def _wait_gathers(isrc, idst, rS, rD, nS, nD, sem):
        pltpu.make_async_copy(srcP.at[isrc], rS, sem).wait()
        pltpu.make_async_copy(dstQ.at[idst], rD, sem).wait()
        pltpu.make_async_copy(nfpk.at[isrc], nS, sem).wait()
        pltpu.make_async_copy(nfpk.at[idst], nD, sem).wait()

    def _wait_scatter(eb, sidx, zb, zsb, sem):
        pltpu.make_async_copy(zb, Ssh.at[sidx], sem).wait()
        pltpu.make_async_copy(zsb, zs_out.at[pl.ds(eb, 128)], sem).wait()

    def _copy_sidx(isrc, sidx):
        for g in range(8):
            off = g * 16
            sidx[pl.ds(off, 16)] = isrc[pl.ds(off, 16)]

    def _compute_chunk(rS, rD, nS, nD, zb, zsb):
        for g in range(8):
            off = g * 16
            rows = iota + off
            hsc = None
            for j in range(8):
                colj = jnp.full((16,), j, i32)
                ps = plsc.load_gather(rS, [rows, colj])
                qd = plsc.load_gather(rD, [rows, colj])
                rj = jnp.maximum(ps + qd + c0s[j], 0.0)
                term = rj * w4s[j]
                hsc = term if hsc is None else hsc + term
            hsc = hsc + b4_s
            a = nS[pl.ds(off, 16)]
            b = nD[pl.ds(off, 16)]
            freeb = (a & 1) == 1
            selfloop = (lax.shift_right_logical(a, 1)
                        == lax.shift_right_logical(b, 1))
            enabled = selfloop != freeb
            lg = jnp.where(enabled, jnp.maximum(hsc, 0.0), 0.0)
            z = jnp.exp(lg)
            zs = jnp.where(selfloop, -z, z)
            zb[pl.ds(off, 16)] = z
            zsb[pl.ds(off, 16)] = zs

    _load_idx(eb0, idxsrc_a, idxdst_a, semi_a)
    _wait_idx(eb0, idxsrc_a, idxdst_a, semi_a)
    _fire_gathers(idxsrc_a, idxdst_a, recS_a, recD_a, nfpk_c, nfd_a, semg_a)
    _load_idx(eb0 + 128, idxsrc_b, idxdst_b, semi_b)

    def _edge_pair(k, carry):
        eb_a = eb0 + k * 256
        eb_b = eb_a + 128

        _wait_idx(eb_b, idxsrc_b, idxdst_b, semi_b)
        _fire_gathers(idxsrc_b, idxdst_b, recS_b, recD_b, nfs_b, nfd_b,
                      semg_b)

        _wait_gathers(idxsrc_a, idxdst_a, recS_a, recD_a, nfpk_c, nfd_a,
                      semg_a)

        _copy_sidx(idxsrc_a, sidx_a)

        @pl.when(k < KMAX - 1)
        def _():
            _load_idx(eb_a + 256, idxsrc_a, idxdst_a, semi_a)

        _compute_chunk(recS_a, recD_a, nfpk_c, nfd_a, zb_a, zsb_a)
        pltpu.sync_copy(zb_a, Ssh.at[sidx_a], add=True)
        pltpu.sync_copy(zsb_a, zs_out.at[pl.ds(eb_a, 128)])

        _wait_gathers(idxsrc_b, idxdst_b, recS_b, recD_b, nfs_b, nfd_b,
                      semg_b)

        _copy_sidx(idxsrc_b, sidx_b)

        @pl.when(k < KMAX - 1)
        def _():
            _load_idx(eb_b + 256, idxsrc_b, idxdst_b, semi_b)

        _compute_chunk(recS_b, recD_b, nfs_b, nfd_b, zb_b, zsb_b)
        pltpu.sync_copy(zb_b, Ssh.at[sidx_b], add=True)
        pltpu.sync_copy(zsb_b, zs_out.at[pl.ds(eb_b, 128)])

        @pl.when(k < KMAX - 1)
        def _():
            _wait_idx(eb_a + 256, idxsrc_a, idxdst_a, semi_a)
            _fire_gathers(idxsrc_a, idxdst_a, recS_a, recD_a, nfpk_c,
                          nfd_a, semg_a)

        return carry

    lax.fori_loop(0, KMAX, _edge_pair, 0)
    plsc.subcore_barrier()

    def _s_out(i, carry):
        nb = s * NPT + i * 128
        pltpu.sync_copy(Ssh.at[pl.ds(nb, 128)], cars1_c)

        @pl.when(c == 0)
        def _():
            pltpu.sync_copy(cars1_c, s0_out.at[pl.ds(nb, 128)])

        @pl.when(c == 1)
        def _():
            pltpu.sync_copy(cars1_c, s1_out.at[pl.ds(nb, 128)])

        return carry

    lax.fori_loop(0, NCHUNKS_NODE, _s_out, 0)


@functools.partial(
    pl.kernel,
    out_type=[
        jax.ShapeDtypeStruct((NC, N_PAD), f32),
        jax.ShapeDtypeStruct((NC, N_PAD), f32),
        jax.ShapeDtypeStruct((N_PAD, EMB), f32),
    ],
    mesh=_MESH,
    compiler_params=_SC_PARAMS,
    scratch_types=[
        pltpu.VMEM_SHARED((N_PAD,), f32),
        pltpu.VMEM_SHARED((N_PAD,), f32),
        pltpu.VMEM_SHARED((N_PAD,), f32),
        pltpu.VMEM((NPT,), f32),
        pltpu.VMEM((NPT,), f32),
        pltpu.VMEM((NPT,), f32),
        pltpu.VMEM((128,), i32),
        pltpu.VMEM((128, EMB), f32),
        pltpu.VMEM((128,), i32),
        pltpu.VMEM((128,), i32),
        pltpu.VMEM((128,), i32),
        pltpu.VMEM((128,), i32),
        pltpu.VMEM((128,), f32),
        pltpu.VMEM((128,), f32),
        pltpu.VMEM((128,), f32),
        pltpu.VMEM((128,), f32),
        pltpu.VMEM((128,), i32),
        pltpu.VMEM((128,), i32),
        pltpu.VMEM((128,), f32),
        pltpu.VMEM((128,), f32),
        pltpu.VMEM((128,), f32),
        pltpu.VMEM((128,), f32),
        pltpu.SemaphoreType.DMA,
        pltpu.SemaphoreType.DMA,
        pltpu.SemaphoreType.DMA,
        pltpu.SemaphoreType.DMA,
        pltpu.SemaphoreType.DMA,
        pltpu.SemaphoreType.DMA,
    ],
)
def _kern_b(src_hbm, dst_hbm, zs_hbm, s0_hbm, s1_hbm, cars1_hbm, nf_hbm,
            embed_hbm,
            cars_part, ent_part, emb_out,
            T_sh, cacc, eacc, buf_a, buf_b, buf_c, nf_c, embrows,
            idxsrc_a, idxdst_a, idxsrc_b, idxdst_b, zsl_a, zsl_b,
            tb_a, tb_b, sdidx_a, sdidx_b, amb_a, amb_b, outb_a, outb_b,
            semi_a, semi_b, semg_a, semg_b, semz_a, semz_b):
    c = lax.axis_index("c")
    s = lax.axis_index("s")
    tile = c * NS + s
    nb = s * NPT

    pltpu.sync_copy(s0_hbm.at[pl.ds(nb, NPT)], buf_a)
    pltpu.sync_copy(s1_hbm.at[pl.ds(nb, NPT)], buf_b)
    pltpu.sync_copy(cars1_hbm.at[pl.ds(nb, NPT)], buf_c)

    def _t_loop(i, carry):
        o = i * 16
        sv = buf_a[pl.ds(o, 16)] + buf_b[pl.ds(o, 16)]
        buf_a[pl.ds(o, 16)] = buf_c[pl.ds(o, 16)] / sv
        return carry

    lax.fori_loop(0, NPT // 16, _t_loop, 0)
    pltpu.sync_copy(buf_a, T_sh.at[pl.ds(nb, NPT)])

    zero16 = jnp.zeros((16,), f32)

    def _zero_loop(i, carry):
        buf_b[pl.ds(i * 16, 16)] = zero16
        return carry

    lax.fori_loop(0, NPT // 16, _zero_loop, 0)
    pltpu.sync_copy(buf_b, cacc.at[pl.ds(nb, NPT)])
    pltpu.sync_copy(buf_b, eacc.at[pl.ds(nb, NPT)])
    plsc.subcore_barrier()

    eb0 = tile * EPT
    KMAX = NCHUNKS_EDGE // 2

    def _load_in(eb, isrc, idst, zsl, sem):
        pltpu.async_copy(src_hbm.at[pl.ds(eb, 128)], isrc, sem)
        pltpu.async_copy(dst_hbm.at[pl.ds(eb, 128)], idst, sem)
        pltpu.async_copy(zs_hbm.at[pl.ds(eb, 128)], zsl, sem)

    def _wait_in(eb, isrc, idst, zsl, sem):
        pltpu.make_async_copy(src_hbm.at[pl.ds(eb, 128)], isrc, sem).wait()
        pltpu.make_async_copy(dst_hbm.at[pl.ds(eb, 128)], idst, sem).wait()
        pltpu.make_async_copy(zs_hbm.at[pl.ds(eb, 128)], zsl, sem).wait()

    def _wait_sc(sdidx, amb, outb, sem):
        pltpu.make_async_copy(amb, cacc.at[sdidx], sem).wait()
        pltpu.make_async_copy(outb, eacc.at[sdidx], sem).wait()

    def _compute2(zsl, tb, amb, outb):
        for g in range(8):
            o = g * 16
            zsv = zsl[pl.ds(o, 16)]
            tv = tb[pl.ds(o, 16)]
            am = jnp.abs(zsv) * tv
            amb[pl.ds(o, 16)] = am
            outb[pl.ds(o, 16)] = jnp.where(zsv < 0.0, 0.0, am)

    def _copy_sdidx(idst, sdidx):
        for g in range(8):
            o = g * 16
            sdidx[pl.ds(o, 16)] = idst[pl.ds(o, 16)]

    _load_in(eb0, idxsrc_a, idxdst_a, zsl_a, semi_a)
    _wait_in(eb0, idxsrc_a, idxdst_a, zsl_a, semi_a)
    pltpu.async_copy(T_sh.at[idxsrc_a], tb_a, semg_a)
    _load_in(eb0 + 128, idxsrc_b, idxdst_b, zsl_b, semi_b)

    def _edge_pair(k, carry):
        eb_a = eb0 + k * 256
        eb_b = eb_a + 128

        _wait_in(eb_b, idxsrc_b, idxdst_b, zsl_b, semi_b)
        pltpu.async_copy(T_sh.at[idxsrc_b], tb_b, semg_b)

        pltpu.make_async_copy(T_sh.at[idxsrc_a], tb_a, semg_a).wait()
        _compute2(zsl_a, tb_a, amb_a, outb_a)
        _copy_sdidx(idxdst_a, sdidx_a)

        @pl.when(k < KMAX - 1)
        def _():
            _load_in(eb_a + 256, idxsrc_a, idxdst_a, zsl_a, semi_a)

        pltpu.sync_copy(amb_a, cacc.at[sdidx_a], add=True)
        pltpu.sync_copy(outb_a, eacc.at[sdidx_a], add=True)

        pltpu.make_async_copy(T_sh.at[idxsrc_b], tb_b, semg_b).wait()
        _compute2(zsl_b, tb_b, amb_b, outb_b)
        _copy_sdidx(idxdst_b, sdidx_b)

        @pl.when(k < KMAX - 1)
        def _():
            _load_in(eb_b + 256, idxsrc_b, idxdst_b, zsl_b, semi_b)

        pltpu.sync_copy(amb_b, cacc.at[sdidx_b], add=True)
        pltpu.sync_copy(outb_b, eacc.at[sdidx_b], add=True)

        @pl.when(k < KMAX - 1)
        def _():
            _wait_in(eb_a + 256, idxsrc_a, idxdst_a, zsl_a, semi_a)
            pltpu.async_copy(T_sh.at[idxsrc_a], tb_a, semg_a)

        return carry

    lax.fori_loop(0, KMAX, _edge_pair, 0)
    plsc.subcore_barrier()

    pltpu.sync_copy(cacc.at[pl.ds(nb, NPT)], buf_a)
    pltpu.sync_copy(buf_a, cars_part.at[c, pl.ds(nb, NPT)])
    pltpu.sync_copy(eacc.at[pl.ds(nb, NPT)], buf_b)
    pltpu.sync_copy(buf_b, ent_part.at[c, pl.ds(nb, NPT)])

    @pl.when(c == 1)
    def _():
        def _emb_chunk(i, carry):
            nbb = s * NPT + i * 128
            pltpu.sync_copy(nf_hbm.at[pl.ds(nbb, 128)], nf_c)
            pltpu.async_copy(embed_hbm.at[nf_c], embrows, semg_a).wait()
            pltpu.sync_copy(embrows, emb_out.at[pl.ds(nbb, 128)])
            return carry

        lax.fori_loop(0, NCHUNKS_NODE, _emb_chunk, 0)


def kernel(edge_index, nfeatures, cars, free, entered, embed,
           W2p, b2p, W2e, b2e, W3, b3, W4, b4):
    src = edge_index[0]
    dst = edge_index[1]
    pad_e = jnp.full((E_PAD - E,), N, i32)
    src_p = jnp.concatenate([src, pad_e])
    dst_p = jnp.concatenate([dst, pad_e])

    pad_n = N_PAD - N
    nf_p = jnp.concatenate([nfeatures, jnp.zeros((pad_n,), i32)])
    cars_p = jnp.concatenate([cars[:, 0], jnp.zeros((pad_n,), f32)])
    free_p = jnp.concatenate([free[:, 0], jnp.zeros((pad_n,), f32)])
    ent_p = jnp.concatenate([entered[:, 0], jnp.zeros((pad_n,), f32)])

    tabP, tabQ, tp, c0v = _fold(embed, W2e, W3, W2p, b2e, b3)

    pp = jnp.concatenate([
        W3[EMB], W3[EMB + 1],
        W3[EMB + 2], c0v[0],
        W4[:, 0],
        jnp.stack([b2p[0], W2p[EMB, 0], b4[0]]),
        jnp.zeros((5,), f32),
    ])

    cars1, s0, s1, zs = _kern_a(
        nf_p, cars_p, free_p, ent_p, tabP, tabQ, tp[:, 0], pp,
        src_p, dst_p)

    cars_part, ent_part, emb_out = _kern_b(
        src_p, dst_p, zs, s0, s1, cars1, nf_p, embed)

    cars_sum, ent_sum = _combine(cars_part, ent_part)

    return (cars_sum[:N, None], emb_out[:N], ent_sum[:N, None])


# speedup vs baseline: 121.0631x; 1.1403x over previous
import functools

import jax
import jax.numpy as jnp
from jax import lax
from jax.experimental import pallas as pl
from jax.experimental.pallas import tpu as pltpu
from jax.experimental.pallas import tpu_sc as plsc

N = 100000
E = 1600000
EMB = 32
VOCAB = 1000

NC = 2
NS = 16
NPT = 6272
N_PAD = NS * NPT
EPT = 50176
E_PAD = NC * NS * EPT
NCHUNKS_NODE = NPT // 128
NCHUNKS_EDGE = EPT // 128

f32 = jnp.float32
i32 = jnp.int32

_SC_PARAMS = pltpu.CompilerParams(needs_layout_passes=False,
                                  use_tc_tiling_on_sc=False)


def _fold_body(embed_ref, W2e_ref, W3_ref, W2p_ref, b2e_ref, b3_ref,
               tabP_ref, tabQ_ref, tp_ref, c0_ref):
    W3t = W3_ref[0:EMB, :]
    A = jnp.dot(W2e_ref[0:EMB, :], W3t, preferred_element_type=f32,
                precision=lax.Precision.HIGHEST)
    B = jnp.dot(W2e_ref[EMB:2 * EMB, :], W3t, preferred_element_type=f32,
                precision=lax.Precision.HIGHEST)
    emb = embed_ref[...]
    tabP_ref[...] = jnp.dot(emb, A, preferred_element_type=f32,
                            precision=lax.Precision.HIGHEST)
    tabQ_ref[...] = jnp.dot(emb, B, preferred_element_type=f32,
                            precision=lax.Precision.HIGHEST)
    tp_ref[...] = jnp.dot(emb, W2p_ref[0:EMB, :], preferred_element_type=f32,
                          precision=lax.Precision.HIGHEST)
    c0_ref[...] = jnp.dot(b2e_ref[...], W3t, preferred_element_type=f32,
                          precision=lax.Precision.HIGHEST) + b3_ref[...]


def _fold(embed, W2e, W3, W2p, b2e, b3):
    return pl.pallas_call(
        _fold_body,
        out_shape=[
            jax.ShapeDtypeStruct((VOCAB, 8), f32),
            jax.ShapeDtypeStruct((VOCAB, 8), f32),
            jax.ShapeDtypeStruct((VOCAB, 1), f32),
            jax.ShapeDtypeStruct((1, 8), f32),
        ],
    )(embed, W2e, W3, W2p, b2e.reshape(1, EMB), b3.reshape(1, 8))


def _combine_body(c_ref, e_ref, co_ref, eo_ref):
    co_ref[...] = c_ref[0, :] + c_ref[1, :]
    eo_ref[...] = e_ref[0, :] + e_ref[1, :]


def _combine(cars_part, ent_part):
    return pl.pallas_call(
        _combine_body,
        out_shape=[
            jax.ShapeDtypeStruct((N_PAD,), f32),
            jax.ShapeDtypeStruct((N_PAD,), f32),
        ],
    )(cars_part, ent_part)


_MESH = plsc.VectorSubcoreMesh(core_axis_name="c", subcore_axis_name="s",
                               num_cores=NC, num_subcores=NS)


@functools.partial(
    pl.kernel,
    out_type=[
        jax.ShapeDtypeStruct((N_PAD,), f32),
        jax.ShapeDtypeStruct((N_PAD,), f32),
        jax.ShapeDtypeStruct((N_PAD,), f32),
        jax.ShapeDtypeStruct((E_PAD,), f32),
    ],
    mesh=_MESH,
    compiler_params=_SC_PARAMS,
    scratch_types=[
        pltpu.VMEM_SHARED((N_PAD, 8), f32),
        pltpu.VMEM_SHARED((N_PAD, 8), f32),
        pltpu.VMEM_SHARED((N_PAD,), i32),
        pltpu.VMEM_SHARED((N_PAD,), f32),
        pltpu.VMEM((48,), f32),
        pltpu.VMEM((128,), i32),
        pltpu.VMEM((128,), f32),
        pltpu.VMEM((128,), f32),
        pltpu.VMEM((128,), f32),
        pltpu.VMEM((128, 8), f32),
        pltpu.VMEM((128, 8), f32),
        pltpu.VMEM((128, 8), f32),
        pltpu.VMEM((128, 8), f32),
        pltpu.VMEM((128,), f32),
        pltpu.VMEM((128,), i32),
        pltpu.VMEM((128,), i32),
        pltpu.VMEM((128,), i32),
        pltpu.VMEM((128,), i32),
        pltpu.VMEM((128,), f32),
        pltpu.VMEM((128,), i32),
        pltpu.VMEM((128,), i32),
        pltpu.VMEM((128,), i32),
        pltpu.VMEM((128,), i32),
        pltpu.VMEM((128,), i32),
        pltpu.VMEM((128,), i32),
        pltpu.VMEM((128,), f32),
        pltpu.VMEM((128,), f32),
        pltpu.VMEM((128,), f32),
        pltpu.VMEM((128,), f32),
        pltpu.VMEM((128,), i32),
        pltpu.VMEM((128,), f32),
        pltpu.VMEM((128,), f32),
        pltpu.VMEM((128,), f32),
        pltpu.VMEM((128, 8), f32),
        pltpu.VMEM((128, 8), f32),
        pltpu.SemaphoreType.DMA,
        pltpu.SemaphoreType.DMA,
        pltpu.SemaphoreType.DMA,
        pltpu.SemaphoreType.DMA,
        pltpu.SemaphoreType.DMA,
        pltpu.SemaphoreType.DMA,
    ],
)
def _kern_a(nf_hbm, cars_hbm, free_hbm, ent_hbm, tabP_hbm, tabQ_hbm, tp_hbm,
            pp_hbm, src_hbm, dst_hbm,
            cars1_out, s0_out, s1_out, zs_out,
            srcP, dstQ, nfpk, Ssh, pp_v, nf_c, cars_c, free_c, ent_c,
            recS_a, recD_a, recS_b, recD_b, tpg, nfpk_c, nfd_a, nfs_b,
            nfd_b, cars1_c, idxsrc_a, idxdst_a, idxsrc_b, idxdst_b,
            sidx_a, sidx_b, zb_a, zb_b, zsb_a, zsb_b,
            nfl_b, carl_b, frel_b, entl_b, recW_s, recW_d,
            semi_a, semi_b, semg_a, semg_b, semz_a, semz_b):
    c = lax.axis_index("c")
    s = lax.axis_index("s")
    tile = c * NS + s
    iota = lax.iota(i32, 16)

    pltpu.sync_copy(pp_hbm, pp_v)
    row0 = pp_v[pl.ds(0, 16)]
    row1 = pp_v[pl.ds(16, 16)]
    row2 = pp_v[pl.ds(32, 16)]
    wc1 = [row0[j] for j in range(8)]
    wc2 = [row0[8 + j] for j in range(8)]
    we = [row1[j] for j in range(8)]
    c0s = [row1[8 + j] for j in range(8)]
    w4s = [row2[j] for j in range(8)]
    b2p_s = row2[8]
    w2pc_s = row2[9]
    b4_s = row2[10]

    zero16 = jnp.zeros((16,), f32)

    def _zfill(i, carry):
        cars1_c[pl.ds(i * 16, 16)] = zero16
        return carry

    lax.fori_loop(0, 8, _zfill, 0)

    def _zero_loop(i, carry):
        pltpu.sync_copy(cars1_c, Ssh.at[pl.ds(s * NPT + i * 128, 128)])
        return carry

    lax.fori_loop(0, NCHUNKS_NODE, _zero_loop, 0)

    nb0 = s * NPT

    def _n_load(nb, nfb, carb, freb, entb, sem):
        pltpu.async_copy(nf_hbm.at[pl.ds(nb, 128)], nfb, sem)
        pltpu.async_copy(cars_hbm.at[pl.ds(nb, 128)], carb, sem)
        pltpu.async_copy(free_hbm.at[pl.ds(nb, 128)], freb, sem)
        pltpu.async_copy(ent_hbm.at[pl.ds(nb, 128)], entb, sem)

    def _n_wait_load(nb, nfb, carb, freb, entb, sem):
        pltpu.make_async_copy(nf_hbm.at[pl.ds(nb, 128)], nfb, sem).wait()
        pltpu.make_async_copy(cars_hbm.at[pl.ds(nb, 128)], carb, sem).wait()
        pltpu.make_async_copy(free_hbm.at[pl.ds(nb, 128)], freb, sem).wait()
        pltpu.make_async_copy(ent_hbm.at[pl.ds(nb, 128)], entb, sem).wait()

    def _n_gather(nfb, tPg, tQg, tpb, sem):
        pltpu.async_copy(tabP_hbm.at[nfb], tPg, sem)
        pltpu.async_copy(tabQ_hbm.at[nfb], tQg, sem)
        pltpu.async_copy(tp_hbm.at[nfb], tpb, sem)

    def _n_wait_gather(nfb, tPg, tQg, tpb, sem):
        pltpu.make_async_copy(tabP_hbm.at[nfb], tPg, sem).wait()
        pltpu.make_async_copy(tabQ_hbm.at[nfb], tQg, sem).wait()
        pltpu.make_async_copy(tp_hbm.at[nfb], tpb, sem).wait()

    def _n_compute(nb, nfb, carb, freb, entb, tPg, tQg, tpb):
        for g in range(8):
            off = g * 16
            rows = iota + off
            nfi = nfb[pl.ds(off, 16)]
            carsv = carb[pl.ds(off, 16)]
            freev = freb[pl.ds(off, 16)]
            entv = entb[pl.ds(off, 16)]
            tpv = tpb[pl.ds(off, 16)]
            parked = tpv + carsv * w2pc_s + b2p_s
            cars1v = jnp.maximum(jnp.maximum(parked, 0.0) + carsv, 0.0)
            for j in range(8):
                colj = jnp.full((16,), j, i32)
                pj = (plsc.load_gather(tPg, [rows, colj])
                      + cars1v * wc1[j] + entv * we[j])
                plsc.store_scatter(recW_s, [rows, colj], pj)
                qj = plsc.load_gather(tQg, [rows, colj]) + cars1v * wc2[j]
                plsc.store_scatter(recW_d, [rows, colj], qj)
            nfpk_c[pl.ds(off, 16)] = nfi * 2 + jnp.where(freev > 0.5, 1, 0)
            cars1_c[pl.ds(off, 16)] = cars1v
        pltpu.sync_copy(recW_s, srcP.at[pl.ds(nb, 128)])
        pltpu.sync_copy(recW_d, dstQ.at[pl.ds(nb, 128)])
        pltpu.sync_copy(nfpk_c, nfpk.at[pl.ds(nb, 128)])

        @pl.when(c == 0)
        def _():
            pltpu.sync_copy(cars1_c, cars1_out.at[pl.ds(nb, 128)])

    _n_load(nb0, nf_c, cars_c, free_c, ent_c, semi_a)
    _n_wait_load(nb0, nf_c, cars_c, free_c, ent_c, semi_a)
    _n_gather(nf_c, recS_a, recD_a, zb_a, semg_a)
    _n_load(nb0 + 128, nfl_b, carl_b, frel_b, entl_b, semi_b)

    def _node_pair(k, carry):
        nb_a = nb0 + k * 256
        nb_b = nb_a + 128

        _n_wait_load(nb_b, nfl_b, carl_b, frel_b, entl_b, semi_b)
        _n_gather(nfl_b, recS_b, recD_b, zb_b, semg_b)

        _n_wait_gather(nf_c, recS_a, recD_a, zb_a, semg_a)
        _n_compute(nb_a, nf_c, cars_c, free_c, ent_c, recS_a, recD_a, zb_a)

        _n_load(nb_a + 256, nf_c, cars_c, free_c, ent_c, semi_a)

        _n_wait_gather(nfl_b, recS_b, recD_b, zb_b, semg_b)
        _n_compute(nb_b, nfl_b, carl_b, frel_b, entl_b, recS_b, recD_b,
                   zb_b)

        @pl.when(k < 23)
        def _():
            _n_load(nb_b + 256, nfl_b, carl_b, frel_b, entl_b, semi_b)

        _n_wait_load(nb_a + 256, nf_c, cars_c, free_c, ent_c, semi_a)
        _n_gather(nf_c, recS_a, recD_a, zb_a, semg_a)
        return carry

    lax.fori_loop(0, (NCHUNKS_NODE - 1) // 2, _node_pair, 0)
    _n_wait_gather(nf_c, recS_a, recD_a, zb_a, semg_a)
    _n_compute(nb0 + (NCHUNKS_NODE - 1) * 128, nf_c, cars_c, free_c,
               ent_c, recS_a, recD_a, zb_a)
    plsc.subcore_barrier()

    eb0 = tile * EPT
    KMAX = NCHUNKS_EDGE // 2

    def _load_idx(eb, isrc, idst, sem):
        pltpu.async_copy(src_hbm.at[pl.ds(eb, 128)], isrc, sem)
        pltpu.async_copy(dst_hbm.at[pl.ds(eb, 128)], idst, sem)

    def _wait_idx(eb, isrc, idst, sem):
        pltpu.make_async_copy(src_hbm.at[pl.ds(eb, 128)], isrc, sem).wait()
        pltpu.make_async_copy(dst_hbm.at[pl.ds(eb, 128)], idst, sem).wait()

    def _fire_gathers(isrc, idst, rS, rD, nS, nD, sem):
        pltpu.async_copy(srcP.at[isrc], rS, sem)
        pltpu.async_copy(dstQ.at[idst], rD, sem)
        pltpu.async_copy(nfpk.at[isrc], nS, sem)
        pltpu.async_copy(nfpk.at[idst], nD, sem)

    def _wait_gathers(isrc, idst, rS, rD, nS, nD, sem):
        pltpu.make_async_copy(srcP.at[isrc], rS, sem).wait()
        pltpu.make_async_copy(dstQ.at[idst], rD, sem).wait()
        pltpu.make_async_copy(nfpk.at[isrc], nS, sem).wait()
        pltpu.make_async_copy(nfpk.at[idst], nD, sem).wait()

    def _wait_scatter(eb, sidx, zb, zsb, sem):
        pltpu.make_async_copy(zb, Ssh.at[sidx], sem).wait()
        pltpu.make_async_copy(zsb, zs_out.at[pl.ds(eb, 128)], sem).wait()

    def _copy_sidx(isrc, sidx):
        for g in range(8):
            off = g * 16
            sidx[pl.ds(off, 16)] = isrc[pl.ds(off, 16)]

    def _compute_chunk(rS, rD, nS, nD, zb, zsb):
        for g in range(8):
            off = g * 16
            rows = iota + off
            hsc = None
            for j in range(8):
                colj = jnp.full((16,), j, i32)
                ps = plsc.load_gather(rS, [rows, colj])
                qd = plsc.load_gather(rD, [rows, colj])
                rj = jnp.maximum(ps + qd + c0s[j], 0.0)
                term = rj * w4s[j]
                hsc = term if hsc is None else hsc + term
            hsc = hsc + b4_s
            a = nS[pl.ds(off, 16)]
            b = nD[pl.ds(off, 16)]
            freeb = (a & 1) == 1
            selfloop = (lax.shift_right_logical(a, 1)
                        == lax.shift_right_logical(b, 1))
            enabled = selfloop != freeb
            lg = jnp.where(enabled, jnp.maximum(hsc, 0.0), 0.0)
            z = jnp.exp(lg)
            zs = jnp.where(selfloop, -z, z)
            zb[pl.ds(off, 16)] = z
            zsb[pl.ds(off, 16)] = zs

    _load_idx(eb0, idxsrc_a, idxdst_a, semi_a)
    _wait_idx(eb0, idxsrc_a, idxdst_a, semi_a)
    _fire_gathers(idxsrc_a, idxdst_a, recS_a, recD_a, nfpk_c, nfd_a, semg_a)
    _load_idx(eb0 + 128, idxsrc_b, idxdst_b, semi_b)

    def _edge_pair(k, carry):
        eb_a = eb0 + k * 256
        eb_b = eb_a + 128

        _wait_idx(eb_b, idxsrc_b, idxdst_b, semi_b)
        _fire_gathers(idxsrc_b, idxdst_b, recS_b, recD_b, nfs_b, nfd_b,
                      semg_b)

        _wait_gathers(idxsrc_a, idxdst_a, recS_a, recD_a, nfpk_c, nfd_a,
                      semg_a)

        _copy_sidx(idxsrc_a, sidx_a)

        @pl.when(k < KMAX - 1)
        def _():
            _load_idx(eb_a + 256, idxsrc_a, idxdst_a, semi_a)

        _compute_chunk(recS_a, recD_a, nfpk_c, nfd_a, zb_a, zsb_a)
        cp_za = pltpu.async_copy(zb_a, Ssh.at[sidx_a], semz_a, add=True)
        cp_wa = pltpu.async_copy(zsb_a, zs_out.at[pl.ds(eb_a, 128)],
                                 semz_b)

        _wait_gathers(idxsrc_b, idxdst_b, recS_b, recD_b, nfs_b, nfd_b,
                      semg_b)

        _copy_sidx(idxsrc_b, sidx_b)

        @pl.when(k < KMAX - 1)
        def _():
            _load_idx(eb_b + 256, idxsrc_b, idxdst_b, semi_b)

        _compute_chunk(recS_b, recD_b, nfs_b, nfd_b, zb_b, zsb_b)
        cp_za.wait()
        cp_wa.wait()
        cp_zb = pltpu.async_copy(zb_b, Ssh.at[sidx_b], semz_a, add=True)
        cp_wb = pltpu.async_copy(zsb_b, zs_out.at[pl.ds(eb_b, 128)],
                                 semz_b)

        @pl.when(k < KMAX - 1)
        def _():
            _wait_idx(eb_a + 256, idxsrc_a, idxdst_a, semi_a)
            _fire_gathers(idxsrc_a, idxdst_a, recS_a, recD_a, nfpk_c,
                          nfd_a, semg_a)

        cp_zb.wait()
        cp_wb.wait()
        return carry

    lax.fori_loop(0, KMAX, _edge_pair, 0)
    plsc.subcore_barrier()

    def _s_out(i, carry):
        nb = s * NPT + i * 128
        pltpu.sync_copy(Ssh.at[pl.ds(nb, 128)], cars1_c)

        @pl.when(c == 0)
        def _():
            pltpu.sync_copy(cars1_c, s0_out.at[pl.ds(nb, 128)])

        @pl.when(c == 1)
        def _():
            pltpu.sync_copy(cars1_c, s1_out.at[pl.ds(nb, 128)])

        return carry

    lax.fori_loop(0, NCHUNKS_NODE, _s_out, 0)


@functools.partial(
    pl.kernel,
    out_type=[
        jax.ShapeDtypeStruct((NC, N_PAD), f32),
        jax.ShapeDtypeStruct((NC, N_PAD), f32),
        jax.ShapeDtypeStruct((N_PAD, EMB), f32),
    ],
    mesh=_MESH,
    compiler_params=_SC_PARAMS,
    scratch_types=[
        pltpu.VMEM_SHARED((N_PAD,), f32),
        pltpu.VMEM_SHARED((N_PAD,), f32),
        pltpu.VMEM_SHARED((N_PAD,), f32),
        pltpu.VMEM((NPT,), f32),
        pltpu.VMEM((NPT,), f32),
        pltpu.VMEM((NPT,), f32),
        pltpu.VMEM((128,), i32),
        pltpu.VMEM((128, EMB), f32),
        pltpu.VMEM((128,), i32),
        pltpu.VMEM((128,), i32),
        pltpu.VMEM((128,), i32),
        pltpu.VMEM((128,), i32),
        pltpu.VMEM((128,), f32),
        pltpu.VMEM((128,), f32),
        pltpu.VMEM((128,), f32),
        pltpu.VMEM((128,), f32),
        pltpu.VMEM((128,), i32),
        pltpu.VMEM((128,), i32),
        pltpu.VMEM((128,), f32),
        pltpu.VMEM((128,), f32),
        pltpu.VMEM((128,), f32),
        pltpu.VMEM((128,), f32),
        pltpu.SemaphoreType.DMA,
        pltpu.SemaphoreType.DMA,
        pltpu.SemaphoreType.DMA,
        pltpu.SemaphoreType.DMA,
        pltpu.SemaphoreType.DMA,
        pltpu.SemaphoreType.DMA,
    ],
)
def _kern_b(src_hbm, dst_hbm, zs_hbm, s0_hbm, s1_hbm, cars1_hbm, nf_hbm,
            embed_hbm,
            cars_part, ent_part, emb_out,
            T_sh, cacc, eacc, buf_a, buf_b, buf_c, nf_c, embrows,
            idxsrc_a, idxdst_a, idxsrc_b, idxdst_b, zsl_a, zsl_b,
            tb_a, tb_b, sdidx_a, sdidx_b, amb_a, amb_b, outb_a, outb_b,
            semi_a, semi_b, semg_a, semg_b, semz_a, semz_b):
    c = lax.axis_index("c")
    s = lax.axis_index("s")
    tile = c * NS + s
    nb = s * NPT

    pltpu.sync_copy(s0_hbm.at[pl.ds(nb, NPT)], buf_a)
    pltpu.sync_copy(s1_hbm.at[pl.ds(nb, NPT)], buf_b)
    pltpu.sync_copy(cars1_hbm.at[pl.ds(nb, NPT)], buf_c)

    def _t_loop(i, carry):
        o = i * 16
        sv = buf_a[pl.ds(o, 16)] + buf_b[pl.ds(o, 16)]
        buf_a[pl.ds(o, 16)] = buf_c[pl.ds(o, 16)] / sv
        return carry

    lax.fori_loop(0, NPT // 16, _t_loop, 0)
    pltpu.sync_copy(buf_a, T_sh.at[pl.ds(nb, NPT)])

    zero16 = jnp.zeros((16,), f32)

    def _zero_loop(i, carry):
        buf_b[pl.ds(i * 16, 16)] = zero16
        return carry

    lax.fori_loop(0, NPT // 16, _zero_loop, 0)
    pltpu.sync_copy(buf_b, cacc.at[pl.ds(nb, NPT)])
    pltpu.sync_copy(buf_b, eacc.at[pl.ds(nb, NPT)])
    plsc.subcore_barrier()

    eb0 = tile * EPT
    KMAX = NCHUNKS_EDGE // 2

    def _load_in(eb, isrc, idst, zsl, sem):
        pltpu.async_copy(src_hbm.at[pl.ds(eb, 128)], isrc, sem)
        pltpu.async_copy(dst_hbm.at[pl.ds(eb, 128)], idst, sem)
        pltpu.async_copy(zs_hbm.at[pl.ds(eb, 128)], zsl, sem)

    def _wait_in(eb, isrc, idst, zsl, sem):
        pltpu.make_async_copy(src_hbm.at[pl.ds(eb, 128)], isrc, sem).wait()
        pltpu.make_async_copy(dst_hbm.at[pl.ds(eb, 128)], idst, sem).wait()
        pltpu.make_async_copy(zs_hbm.at[pl.ds(eb, 128)], zsl, sem).wait()

    def _wait_sc(sdidx, amb, outb, sem):
        pltpu.make_async_copy(amb, cacc.at[sdidx], sem).wait()
        pltpu.make_async_copy(outb, eacc.at[sdidx], sem).wait()

    def _compute2(zsl, tb, amb, outb):
        for g in range(8):
            o = g * 16
            zsv = zsl[pl.ds(o, 16)]
            tv = tb[pl.ds(o, 16)]
            am = jnp.abs(zsv) * tv
            amb[pl.ds(o, 16)] = am
            outb[pl.ds(o, 16)] = jnp.where(zsv < 0.0, 0.0, am)

    def _copy_sdidx(idst, sdidx):
        for g in range(8):
            o = g * 16
            sdidx[pl.ds(o, 16)] = idst[pl.ds(o, 16)]

    _load_in(eb0, idxsrc_a, idxdst_a, zsl_a, semi_a)
    _wait_in(eb0, idxsrc_a, idxdst_a, zsl_a, semi_a)
    pltpu.async_copy(T_sh.at[idxsrc_a], tb_a, semg_a)
    _load_in(eb0 + 128, idxsrc_b, idxdst_b, zsl_b, semi_b)

    def _edge_pair(k, carry):
        eb_a = eb0 + k * 256
        eb_b = eb_a + 128

        _wait_in(eb_b, idxsrc_b, idxdst_b, zsl_b, semi_b)
        pltpu.async_copy(T_sh.at[idxsrc_b], tb_b, semg_b)

        pltpu.make_async_copy(T_sh.at[idxsrc_a], tb_a, semg_a).wait()
        _compute2(zsl_a, tb_a, amb_a, outb_a)
        _copy_sdidx(idxdst_a, sdidx_a)

        @pl.when(k < KMAX - 1)
        def _():
            _load_in(eb_a + 256, idxsrc_a, idxdst_a, zsl_a, semi_a)

        cp_ca = pltpu.async_copy(amb_a, cacc.at[sdidx_a], semz_a, add=True)
        cp_ea = pltpu.async_copy(outb_a, eacc.at[sdidx_a], semz_a, add=True)

        pltpu.make_async_copy(T_sh.at[idxsrc_b], tb_b, semg_b).wait()
        _compute2(zsl_b, tb_b, amb_b, outb_b)
        _copy_sdidx(idxdst_b, sdidx_b)

        @pl.when(k < KMAX - 1)
        def _():
            _load_in(eb_b + 256, idxsrc_b, idxdst_b, zsl_b, semi_b)

        cp_ca.wait()
        cp_ea.wait()
        cp_cb = pltpu.async_copy(amb_b, cacc.at[sdidx_b], semz_b, add=True)
        cp_eb = pltpu.async_copy(outb_b, eacc.at[sdidx_b], semz_b, add=True)

        @pl.when(k < KMAX - 1)
        def _():
            _wait_in(eb_a + 256, idxsrc_a, idxdst_a, zsl_a, semi_a)
            pltpu.async_copy(T_sh.at[idxsrc_a], tb_a, semg_a)

        cp_cb.wait()
        cp_eb.wait()
        return carry

    lax.fori_loop(0, KMAX, _edge_pair, 0)
    plsc.subcore_barrier()

    pltpu.sync_copy(cacc.at[pl.ds(nb, NPT)], buf_a)
    pltpu.sync_copy(buf_a, cars_part.at[c, pl.ds(nb, NPT)])
    pltpu.sync_copy(eacc.at[pl.ds(nb, NPT)], buf_b)
    pltpu.sync_copy(buf_b, ent_part.at[c, pl.ds(nb, NPT)])

    @pl.when(c == 1)
    def _():
        def _emb_chunk(i, carry):
            nbb = s * NPT + i * 128
            pltpu.sync_copy(nf_hbm.at[pl.ds(nbb, 128)], nf_c)
            pltpu.async_copy(embed_hbm.at[nf_c], embrows, semg_a).wait()
            pltpu.sync_copy(embrows, emb_out.at[pl.ds(nbb, 128)])
            return carry

        lax.fori_loop(0, NCHUNKS_NODE, _emb_chunk, 0)


def kernel(edge_index, nfeatures, cars, free, entered, embed,
           W2p, b2p, W2e, b2e, W3, b3, W4, b4):
    src = edge_index[0]
    dst = edge_index[1]
    pad_e = jnp.full((E_PAD - E,), N, i32)
    src_p = jnp.concatenate([src, pad_e])
    dst_p = jnp.concatenate([dst, pad_e])

    pad_n = N_PAD - N
    nf_p = jnp.concatenate([nfeatures, jnp.zeros((pad_n,), i32)])
    cars_p = jnp.concatenate([cars[:, 0], jnp.zeros((pad_n,), f32)])
    free_p = jnp.concatenate([free[:, 0], jnp.zeros((pad_n,), f32)])
    ent_p = jnp.concatenate([entered[:, 0], jnp.zeros((pad_n,), f32)])

    tabP, tabQ, tp, c0v = _fold(embed, W2e, W3, W2p, b2e, b3)

    pp = jnp.concatenate([
        W3[EMB], W3[EMB + 1],
        W3[EMB + 2], c0v[0],
        W4[:, 0],
        jnp.stack([b2p[0], W2p[EMB, 0], b4[0]]),
        jnp.zeros((5,), f32),
    ])

    cars1, s0, s1, zs = _kern_a(
        nf_p, cars_p, free_p, ent_p, tabP, tabQ, tp[:, 0], pp,
        src_p, dst_p)

    cars_part, ent_part, emb_out = _kern_b(
        src_p, dst_p, zs, s0, s1, cars1, nf_p, embed)

    cars_sum, ent_sum = _combine(cars_part, ent_part)

    return (cars_sum[:N, None], emb_out[:N], ent_sum[:N, None])
